# Initial kernel scaffold; baseline (speedup 1.0000x reference)
import jax
import jax.numpy as jnp
from jax.experimental import pallas as pl


def kernel(x, edge_index, W1, b1, W2, b2, Wlin, blin):
    raise NotImplementedError("write your pallas kernel here")


# speedup vs baseline: 28.3694x; 28.3694x over previous
import jax
import jax.numpy as jnp
from jax import lax
from jax.experimental import pallas as pl
from jax.experimental.pallas import tpu as pltpu
from jax.experimental.pallas import tpu_sc as plsc

N = 10000
E = 320000
F_IN = 128
F_MID = 16

NC = 2
NS = 16
NW = NC * NS
CHUNK = 128
CPW = 80
E_PAD = NW * CPW * CHUNK
N_PAD = 10240
ROWS_PT = N_PAD // NS

_mesh = plsc.VectorSubcoreMesh(core_axis_name="c", subcore_axis_name="s")


def _make_sc_pass(gather: bool):
  scratch = [
      pltpu.VMEM((CPW, CHUNK), jnp.int32),
      pltpu.VMEM((CHUNK, F_MID), jnp.float32),
      pltpu.SemaphoreType.DMA,
      pltpu.VMEM_SHARED((N_PAD, F_MID), jnp.float32),
  ]
  if gather:
    scratch.insert(0, pltpu.VMEM((CPW, CHUNK), jnp.int32))

  def body(*refs):
    if gather:
      (table_h, src_h, dst_h, zrows_h, out_h,
       src_v, dst_v, rows_v, sem, acc) = refs
    else:
      (ones_h, dst_h, zrows_h, out_h,
       dst_v, rows_v, sem, acc) = refs
    c = lax.axis_index("c")
    s = lax.axis_index("s")
    wid = s * NC + c
    pltpu.sync_copy(zrows_h, acc.at[pl.ds(s * ROWS_PT, ROWS_PT)])
    pltpu.sync_copy(dst_h.at[pl.ds(wid * CPW, CPW)], dst_v)
    if gather:
      pltpu.sync_copy(src_h.at[pl.ds(wid * CPW, CPW)], src_v)
    else:
      pltpu.sync_copy(ones_h, rows_v)
    plsc.subcore_barrier()

    @pl.loop(0, CPW)
    def _(j):
      if gather:
        pltpu.async_copy(table_h.at[src_v.at[j]], rows_v, sem).wait()
      pltpu.sync_copy(rows_v, acc.at[dst_v.at[j]], add=True)

    plsc.subcore_barrier()
    pltpu.sync_copy(acc.at[pl.ds(s * ROWS_PT, ROWS_PT)],
                    out_h.at[c, pl.ds(s * ROWS_PT, ROWS_PT)])

  return pl.kernel(
      body,
      out_type=jax.ShapeDtypeStruct((NC, N_PAD, F_MID), jnp.float32),
      mesh=_mesh,
      scratch_types=scratch,
      compiler_params=pltpu.CompilerParams(use_tc_tiling_on_sc=False),
  )


_sc_deg = _make_sc_pass(gather=False)
_sc_agg = _make_sc_pass(gather=True)

_RB = 2000
_GRID = N // _RB


def _tc1(x, w1, degp):
  def body(x_ref, w_ref, p_ref, hs_ref, dv_ref):
    deg = p_ref[0] + p_ref[1] + 1.0
    dinv = lax.rsqrt(deg)
    h = jnp.dot(x_ref[...], w_ref[...], preferred_element_type=jnp.float32)
    hs_ref[...] = dinv * h
    dv_ref[...] = dinv

  return pl.pallas_call(
      body,
      grid=(_GRID,),
      in_specs=[
          pl.BlockSpec((_RB, F_IN), lambda i: (i, 0)),
          pl.BlockSpec((F_IN, F_MID), lambda i: (0, 0)),
          pl.BlockSpec((NC, _RB, F_MID), lambda i: (0, i, 0)),
      ],
      out_specs=[pl.BlockSpec((_RB, F_MID), lambda i: (i, 0))] * 2,
      out_shape=[jax.ShapeDtypeStruct((N, F_MID), jnp.float32)] * 2,
  )(x, w1, degp)


def _tc2(p, hs1, dinv, b1b):
  def body(p_ref, hs_ref, dv_ref, b_ref, o_ref):
    z = dv_ref[...] * (p_ref[0] + p_ref[1] + hs_ref[...]) + b_ref[0:1, :]
    o_ref[...] = dv_ref[...] * jnp.maximum(z, 0.0)

  return pl.pallas_call(
      body,
      grid=(_GRID,),
      in_specs=[
          pl.BlockSpec((NC, _RB, F_MID), lambda i: (0, i, 0)),
          pl.BlockSpec((_RB, F_MID), lambda i: (i, 0)),
          pl.BlockSpec((_RB, F_MID), lambda i: (i, 0)),
          pl.BlockSpec((8, F_MID), lambda i: (0, 0)),
      ],
      out_specs=pl.BlockSpec((_RB, F_MID), lambda i: (i, 0)),
      out_shape=jax.ShapeDtypeStruct((N, F_MID), jnp.float32),
  )(p, hs1, dinv, b1b)


def _tc3(p, hs2, dinv, w2, b2b, wlin, blinb):
  def body(p_ref, hs_ref, dv_ref, w2_ref, b2_ref, wl_ref, bl_ref, o_ref):
    agg = dv_ref[...] * (p_ref[0] + p_ref[1] + hs_ref[...])
    h2 = jnp.dot(agg, w2_ref[...], preferred_element_type=jnp.float32)
    h2 = jnp.maximum(h2 + b2_ref[0:1, :], 0.0)
    o_ref[...] = (jnp.dot(h2, wl_ref[...], preferred_element_type=jnp.float32)
                  + bl_ref[0:1, 0:1])

  return pl.pallas_call(
      body,
      grid=(_GRID,),
      in_specs=[
          pl.BlockSpec((NC, _RB, F_MID), lambda i: (0, i, 0)),
          pl.BlockSpec((_RB, F_MID), lambda i: (i, 0)),
          pl.BlockSpec((_RB, F_MID), lambda i: (i, 0)),
          pl.BlockSpec((F_MID, F_IN), lambda i: (0, 0)),
          pl.BlockSpec((8, F_IN), lambda i: (0, 0)),
          pl.BlockSpec((F_IN, 1), lambda i: (0, 0)),
          pl.BlockSpec((8, 8), lambda i: (0, 0)),
      ],
      out_specs=pl.BlockSpec((_RB, 1), lambda i: (i, 0)),
      out_shape=jax.ShapeDtypeStruct((N, 1), jnp.float32),
  )(p, hs2, dinv, w2, b2b, wlin, blinb)


@jax.jit
def kernel(x, edge_index, W1, b1, W2, b2, Wlin, blin):
  src = edge_index[0].astype(jnp.int32)
  dst = edge_index[1].astype(jnp.int32)
  pad = E_PAD - E
  src2d = jnp.concatenate(
      [src, jnp.zeros((pad,), jnp.int32)]).reshape(NW * CPW, CHUNK)
  dst2d = jnp.concatenate(
      [dst, jnp.full((pad,), N_PAD - 1, jnp.int32)]).reshape(NW * CPW, CHUNK)
  ones_rows = jnp.ones((CHUNK, F_MID), jnp.float32)
  zrows = jnp.zeros((ROWS_PT, F_MID), jnp.float32)
  b1b = jnp.broadcast_to(b1.reshape(1, F_MID), (8, F_MID))
  b2b = jnp.broadcast_to(b2.reshape(1, F_IN), (8, F_IN))
  blinb = jnp.broadcast_to(blin.reshape(1, 1), (8, 8))

  degp = _sc_deg(ones_rows, dst2d, zrows)
  hs1, dinv = _tc1(x, W1, degp)
  p1 = _sc_agg(hs1, src2d, dst2d, zrows)
  hs2 = _tc2(p1, hs1, dinv, b1b)
  p2 = _sc_agg(hs2, src2d, dst2d, zrows)
  return _tc3(p2, hs2, dinv, W2, b2b, Wlin, blinb)


# speedup vs baseline: 33.5002x; 1.1809x over previous
import jax
import jax.numpy as jnp
from jax import lax
from jax.experimental import pallas as pl
from jax.experimental.pallas import tpu as pltpu
from jax.experimental.pallas import tpu_sc as plsc

N = 10000
E = 320000
F_IN = 128
F_MID = 16

NC = 2
NS = 16
NW = NC * NS
CHUNK = 128
CPW = 80
E_PAD = NW * CPW * CHUNK
N_PAD = 10240
ROWS_PT = N_PAD // NS

_mesh = plsc.VectorSubcoreMesh(core_axis_name="c", subcore_axis_name="s")


def _make_sc_pass(gather: bool):
  nbuf = 8
  scratch = [
      pltpu.VMEM((CPW, CHUNK), jnp.int32),
      pltpu.VMEM((CHUNK, F_MID), jnp.float32),
      pltpu.SemaphoreType.DMA,
      pltpu.VMEM_SHARED((N_PAD, F_MID), jnp.float32),
  ]
  if gather:
    scratch.insert(0, pltpu.VMEM((CPW, CHUNK), jnp.int32))
    scratch += [pltpu.VMEM((CHUNK, F_MID), jnp.float32) for _ in range(nbuf)]

  def body(*refs):
    if gather:
      (table_h, src_h, dst_h, zrows_h, out_h,
       src_v, dst_v, rows_v, sem, acc, *bufs) = refs
    else:
      (ones_h, dst_h, zrows_h, out_h,
       dst_v, rows_v, sem, acc) = refs
    c = lax.axis_index("c")
    s = lax.axis_index("s")
    wid = s * NC + c
    pltpu.sync_copy(zrows_h, acc.at[pl.ds(s * ROWS_PT, ROWS_PT)])
    pltpu.sync_copy(dst_h.at[pl.ds(wid * CPW, CPW)], dst_v)
    if gather:
      pltpu.sync_copy(src_h.at[pl.ds(wid * CPW, CPW)], src_v)
    else:
      pltpu.sync_copy(ones_h, rows_v)
    plsc.subcore_barrier()

    if gather:
      for b in range(nbuf):
        pltpu.async_copy(table_h.at[src_v.at[b]], bufs[b], sem)

      @pl.loop(0, CPW, step=nbuf)
      def _(jo):
        for b in range(nbuf):
          pltpu.make_async_copy(table_h.at[src_v.at[0]], bufs[b], sem).wait()
        for b in range(nbuf):
          pltpu.sync_copy(bufs[b], acc.at[dst_v.at[jo + b]], add=True)
        for b in range(nbuf):
          jn = jo + b + nbuf
          jn = lax.select(jn >= CPW, jn - CPW, jn)
          pltpu.async_copy(table_h.at[src_v.at[jn]], bufs[b], sem)
      for b in range(nbuf):
        pltpu.make_async_copy(table_h.at[src_v.at[0]], bufs[b], sem).wait()
    else:
      @pl.loop(0, CPW)
      def _(j):
        pltpu.async_copy(rows_v, acc.at[dst_v.at[j]], sem, add=True)

      @pl.loop(0, CPW)
      def _(j):
        pltpu.make_async_copy(rows_v, acc.at[dst_v.at[0]], sem).wait()

    plsc.subcore_barrier()
    pltpu.sync_copy(acc.at[pl.ds(s * ROWS_PT, ROWS_PT)],
                    out_h.at[c, pl.ds(s * ROWS_PT, ROWS_PT)])

  return pl.kernel(
      body,
      out_type=jax.ShapeDtypeStruct((NC, N_PAD, F_MID), jnp.float32),
      mesh=_mesh,
      scratch_types=scratch,
      compiler_params=pltpu.CompilerParams(use_tc_tiling_on_sc=False),
  )


_sc_deg = _make_sc_pass(gather=False)
_sc_agg = _make_sc_pass(gather=True)

_RB = 2000
_GRID = N // _RB


def _tc1(x, w1, degp):
  def body(x_ref, w_ref, p_ref, hs_ref, dv_ref):
    deg = p_ref[0] + p_ref[1] + 1.0
    dinv = lax.rsqrt(deg)
    h = jnp.dot(x_ref[...], w_ref[...], preferred_element_type=jnp.float32)
    hs_ref[...] = dinv * h
    dv_ref[...] = dinv

  return pl.pallas_call(
      body,
      grid=(_GRID,),
      in_specs=[
          pl.BlockSpec((_RB, F_IN), lambda i: (i, 0)),
          pl.BlockSpec((F_IN, F_MID), lambda i: (0, 0)),
          pl.BlockSpec((NC, _RB, F_MID), lambda i: (0, i, 0)),
      ],
      out_specs=[pl.BlockSpec((_RB, F_MID), lambda i: (i, 0))] * 2,
      out_shape=[jax.ShapeDtypeStruct((N, F_MID), jnp.float32)] * 2,
  )(x, w1, degp)


def _tc2(p, hs1, dinv, b1b):
  def body(p_ref, hs_ref, dv_ref, b_ref, o_ref):
    z = dv_ref[...] * (p_ref[0] + p_ref[1] + hs_ref[...]) + b_ref[0:1, :]
    o_ref[...] = dv_ref[...] * jnp.maximum(z, 0.0)

  return pl.pallas_call(
      body,
      grid=(_GRID,),
      in_specs=[
          pl.BlockSpec((NC, _RB, F_MID), lambda i: (0, i, 0)),
          pl.BlockSpec((_RB, F_MID), lambda i: (i, 0)),
          pl.BlockSpec((_RB, F_MID), lambda i: (i, 0)),
          pl.BlockSpec((8, F_MID), lambda i: (0, 0)),
      ],
      out_specs=pl.BlockSpec((_RB, F_MID), lambda i: (i, 0)),
      out_shape=jax.ShapeDtypeStruct((N, F_MID), jnp.float32),
  )(p, hs1, dinv, b1b)


def _tc3(p, hs2, dinv, w2, b2b, wlin, blinb):
  def body(p_ref, hs_ref, dv_ref, w2_ref, b2_ref, wl_ref, bl_ref, o_ref):
    agg = dv_ref[...] * (p_ref[0] + p_ref[1] + hs_ref[...])
    h2 = jnp.dot(agg, w2_ref[...], preferred_element_type=jnp.float32)
    h2 = jnp.maximum(h2 + b2_ref[0:1, :], 0.0)
    o_ref[...] = (jnp.dot(h2, wl_ref[...], preferred_element_type=jnp.float32)
                  + bl_ref[0:1, 0:1])

  return pl.pallas_call(
      body,
      grid=(_GRID,),
      in_specs=[
          pl.BlockSpec((NC, _RB, F_MID), lambda i: (0, i, 0)),
          pl.BlockSpec((_RB, F_MID), lambda i: (i, 0)),
          pl.BlockSpec((_RB, F_MID), lambda i: (i, 0)),
          pl.BlockSpec((F_MID, F_IN), lambda i: (0, 0)),
          pl.BlockSpec((8, F_IN), lambda i: (0, 0)),
          pl.BlockSpec((F_IN, 1), lambda i: (0, 0)),
          pl.BlockSpec((8, 8), lambda i: (0, 0)),
      ],
      out_specs=pl.BlockSpec((_RB, 1), lambda i: (i, 0)),
      out_shape=jax.ShapeDtypeStruct((N, 1), jnp.float32),
  )(p, hs2, dinv, w2, b2b, wlin, blinb)


@jax.jit
def kernel(x, edge_index, W1, b1, W2, b2, Wlin, blin):
  src = edge_index[0].astype(jnp.int32)
  dst = edge_index[1].astype(jnp.int32)
  pad = E_PAD - E
  src2d = jnp.concatenate(
      [src, jnp.zeros((pad,), jnp.int32)]).reshape(NW * CPW, CHUNK)
  dst2d = jnp.concatenate(
      [dst, jnp.full((pad,), N_PAD - 1, jnp.int32)]).reshape(NW * CPW, CHUNK)
  ones_rows = jnp.ones((CHUNK, F_MID), jnp.float32)
  zrows = jnp.zeros((ROWS_PT, F_MID), jnp.float32)
  b1b = jnp.broadcast_to(b1.reshape(1, F_MID), (8, F_MID))
  b2b = jnp.broadcast_to(b2.reshape(1, F_IN), (8, F_IN))
  blinb = jnp.broadcast_to(blin.reshape(1, 1), (8, 8))

  degp = _sc_deg(ones_rows, dst2d, zrows)
  hs1, dinv = _tc1(x, W1, degp)
  p1 = _sc_agg(hs1, src2d, dst2d, zrows)
  hs2 = _tc2(p1, hs1, dinv, b1b)
  p2 = _sc_agg(hs2, src2d, dst2d, zrows)
  return _tc3(p2, hs2, dinv, W2, b2b, Wlin, blinb)


# speedup vs baseline: 33.5159x; 1.0005x over previous
import jax
import jax.numpy as jnp
from jax import lax
from jax.experimental import pallas as pl
from jax.experimental.pallas import tpu as pltpu
from jax.experimental.pallas import tpu_sc as plsc

N = 10000
E = 320000
F_IN = 128
F_MID = 16

NC = 2
NS = 16
NW = NC * NS
CHUNK = 512
CPW = 20
E_PAD = NW * CPW * CHUNK
N_PAD = 10240
ROWS_PT = N_PAD // NS

_mesh = plsc.VectorSubcoreMesh(core_axis_name="c", subcore_axis_name="s")


def _make_sc_pass(gather: bool):
  nbuf = 4
  scratch = [
      pltpu.VMEM((CPW, CHUNK), jnp.int32),
      pltpu.VMEM((CHUNK, F_MID), jnp.float32),
      pltpu.SemaphoreType.DMA,
      pltpu.VMEM_SHARED((N_PAD, F_MID), jnp.float32),
  ]
  if gather:
    scratch.insert(0, pltpu.VMEM((CPW, CHUNK), jnp.int32))
    scratch += [pltpu.VMEM((CHUNK, F_MID), jnp.float32) for _ in range(nbuf)]

  def body(*refs):
    if gather:
      (table_h, src_h, dst_h, zrows_h, out_h,
       src_v, dst_v, rows_v, sem, acc, *bufs) = refs
    else:
      (ones_h, dst_h, zrows_h, out_h,
       dst_v, rows_v, sem, acc) = refs
    c = lax.axis_index("c")
    s = lax.axis_index("s")
    wid = s * NC + c
    pltpu.sync_copy(zrows_h, acc.at[pl.ds(s * ROWS_PT, ROWS_PT)])
    pltpu.sync_copy(dst_h.at[wid], dst_v)
    if gather:
      pltpu.sync_copy(src_h.at[wid], src_v)
    else:
      pltpu.sync_copy(ones_h, rows_v)
    plsc.subcore_barrier()

    if gather:
      for b in range(nbuf):
        pltpu.async_copy(table_h.at[src_v.at[b]], bufs[b], sem)

      @pl.loop(0, CPW, step=nbuf)
      def _(jo):
        for b in range(nbuf):
          pltpu.make_async_copy(table_h.at[src_v.at[0]], bufs[b], sem).wait()
        for b in range(nbuf):
          pltpu.sync_copy(bufs[b], acc.at[dst_v.at[jo + b]], add=True)
        for b in range(nbuf):
          jn = jo + b + nbuf
          jn = lax.select(jn >= CPW, jn - CPW, jn)
          pltpu.async_copy(table_h.at[src_v.at[jn]], bufs[b], sem)
      for b in range(nbuf):
        pltpu.make_async_copy(table_h.at[src_v.at[0]], bufs[b], sem).wait()
    else:
      @pl.loop(0, CPW)
      def _(j):
        pltpu.async_copy(rows_v, acc.at[dst_v.at[j]], sem, add=True)

      @pl.loop(0, CPW)
      def _(j):
        pltpu.make_async_copy(rows_v, acc.at[dst_v.at[0]], sem).wait()

    plsc.subcore_barrier()
    pltpu.sync_copy(acc.at[pl.ds(s * ROWS_PT, ROWS_PT)],
                    out_h.at[c, pl.ds(s * ROWS_PT, ROWS_PT)])

  return pl.kernel(
      body,
      out_type=jax.ShapeDtypeStruct((NC, N_PAD, F_MID), jnp.float32),
      mesh=_mesh,
      scratch_types=scratch,
      compiler_params=pltpu.CompilerParams(use_tc_tiling_on_sc=False),
  )


_sc_deg = _make_sc_pass(gather=False)
_sc_agg = _make_sc_pass(gather=True)

_RB = 2000
_GRID = N // _RB


def _tc1(x, w1, degp):
  def body(x_ref, w_ref, p_ref, hs_ref, dv_ref):
    deg = p_ref[0] + p_ref[1] + 1.0
    dinv = lax.rsqrt(deg)
    h = jnp.dot(x_ref[...], w_ref[...], preferred_element_type=jnp.float32)
    hs_ref[...] = dinv * h
    dv_ref[...] = dinv

  return pl.pallas_call(
      body,
      grid=(_GRID,),
      in_specs=[
          pl.BlockSpec((_RB, F_IN), lambda i: (i, 0)),
          pl.BlockSpec((F_IN, F_MID), lambda i: (0, 0)),
          pl.BlockSpec((NC, _RB, F_MID), lambda i: (0, i, 0)),
      ],
      out_specs=[pl.BlockSpec((_RB, F_MID), lambda i: (i, 0))] * 2,
      out_shape=[jax.ShapeDtypeStruct((N, F_MID), jnp.float32)] * 2,
  )(x, w1, degp)


def _tc2(p, hs1, dinv, b1b):
  def body(p_ref, hs_ref, dv_ref, b_ref, o_ref):
    z = dv_ref[...] * (p_ref[0] + p_ref[1] + hs_ref[...]) + b_ref[0:1, :]
    o_ref[...] = dv_ref[...] * jnp.maximum(z, 0.0)

  return pl.pallas_call(
      body,
      grid=(_GRID,),
      in_specs=[
          pl.BlockSpec((NC, _RB, F_MID), lambda i: (0, i, 0)),
          pl.BlockSpec((_RB, F_MID), lambda i: (i, 0)),
          pl.BlockSpec((_RB, F_MID), lambda i: (i, 0)),
          pl.BlockSpec((8, F_MID), lambda i: (0, 0)),
      ],
      out_specs=pl.BlockSpec((_RB, F_MID), lambda i: (i, 0)),
      out_shape=jax.ShapeDtypeStruct((N, F_MID), jnp.float32),
  )(p, hs1, dinv, b1b)


def _tc3(p, hs2, dinv, w2, b2b, wlin, blinb):
  def body(p_ref, hs_ref, dv_ref, w2_ref, b2_ref, wl_ref, bl_ref, o_ref):
    agg = dv_ref[...] * (p_ref[0] + p_ref[1] + hs_ref[...])
    h2 = jnp.dot(agg, w2_ref[...], preferred_element_type=jnp.float32)
    h2 = jnp.maximum(h2 + b2_ref[0:1, :], 0.0)
    o_ref[...] = (jnp.dot(h2, wl_ref[...], preferred_element_type=jnp.float32)
                  + bl_ref[0:1, 0:1])

  return pl.pallas_call(
      body,
      grid=(_GRID,),
      in_specs=[
          pl.BlockSpec((NC, _RB, F_MID), lambda i: (0, i, 0)),
          pl.BlockSpec((_RB, F_MID), lambda i: (i, 0)),
          pl.BlockSpec((_RB, F_MID), lambda i: (i, 0)),
          pl.BlockSpec((F_MID, F_IN), lambda i: (0, 0)),
          pl.BlockSpec((8, F_IN), lambda i: (0, 0)),
          pl.BlockSpec((F_IN, 1), lambda i: (0, 0)),
          pl.BlockSpec((8, 8), lambda i: (0, 0)),
      ],
      out_specs=pl.BlockSpec((_RB, 1), lambda i: (i, 0)),
      out_shape=jax.ShapeDtypeStruct((N, 1), jnp.float32),
  )(p, hs2, dinv, w2, b2b, wlin, blinb)


@jax.jit
def kernel(x, edge_index, W1, b1, W2, b2, Wlin, blin):
  src = edge_index[0].astype(jnp.int32)
  dst = edge_index[1].astype(jnp.int32)
  pad = E_PAD - E
  src2d = jnp.concatenate(
      [src, jnp.zeros((pad,), jnp.int32)]).reshape(NW, CPW, CHUNK)
  dst2d = jnp.concatenate(
      [dst, jnp.full((pad,), N_PAD - 1, jnp.int32)]).reshape(NW, CPW, CHUNK)
  ones_rows = jnp.ones((CHUNK, F_MID), jnp.float32)
  zrows = jnp.zeros((ROWS_PT, F_MID), jnp.float32)
  b1b = jnp.broadcast_to(b1.reshape(1, F_MID), (8, F_MID))
  b2b = jnp.broadcast_to(b2.reshape(1, F_IN), (8, F_IN))
  blinb = jnp.broadcast_to(blin.reshape(1, 1), (8, 8))

  degp = _sc_deg(ones_rows, dst2d, zrows)
  hs1, dinv = _tc1(x, W1, degp)
  p1 = _sc_agg(hs1, src2d, dst2d, zrows)
  hs2 = _tc2(p1, hs1, dinv, b1b)
  p2 = _sc_agg(hs2, src2d, dst2d, zrows)
  return _tc3(p2, hs2, dinv, W2, b2b, Wlin, blinb)


# speedup vs baseline: 50.8185x; 1.5162x over previous
import jax
import jax.numpy as jnp
from jax import lax
from jax.experimental import pallas as pl
from jax.experimental.pallas import tpu as pltpu
from jax.experimental.pallas import tpu_sc as plsc

N = 10000
E = 320000
F_IN = 128
F_MID = 16

NC = 2
NS = 16
NW = NC * NS
CHUNK = 512
CPW = 20
E_PAD = NW * CPW * CHUNK
N_PAD = 10240
ROWS_PT = N_PAD // NS

_mesh = plsc.VectorSubcoreMesh(core_axis_name="c", subcore_axis_name="s")


def _make_sc_pass(gather: bool):
  nbuf = 4
  scratch = [
      pltpu.VMEM((CPW, CHUNK), jnp.int32),
      pltpu.VMEM((CHUNK, F_MID), jnp.float32),
      pltpu.SemaphoreType.DMA,
      pltpu.VMEM_SHARED((N_PAD, F_MID), jnp.float32),
  ]
  if gather:
    scratch.insert(0, pltpu.VMEM((CPW, CHUNK), jnp.int32))
    scratch += [pltpu.VMEM((CHUNK, F_MID), jnp.float32) for _ in range(nbuf)]
    scratch.append(pltpu.VMEM_SHARED((N_PAD, F_MID), jnp.float32))

  def body(*refs):
    if gather:
      (table_h, src_h, dst_h, zrows_h, out_h,
       src_v, dst_v, rows_v, sem, acc, *bufs, tbl_s) = refs
    else:
      (ones_h, dst_h, zrows_h, out_h,
       dst_v, rows_v, sem, acc) = refs
    c = lax.axis_index("c")
    s = lax.axis_index("s")
    wid = s * NC + c
    pltpu.sync_copy(zrows_h, acc.at[pl.ds(s * ROWS_PT, ROWS_PT)])
    pltpu.sync_copy(dst_h.at[wid], dst_v)
    if gather:
      pltpu.sync_copy(src_h.at[wid], src_v)
      pltpu.sync_copy(table_h.at[pl.ds(s * ROWS_PT, ROWS_PT)],
                      tbl_s.at[pl.ds(s * ROWS_PT, ROWS_PT)])
    else:
      pltpu.sync_copy(ones_h, rows_v)
    plsc.subcore_barrier()

    if gather:
      for b in range(nbuf):
        pltpu.async_copy(tbl_s.at[src_v.at[b]], bufs[b], sem)

      @pl.loop(0, CPW, step=nbuf)
      def _(jo):
        for b in range(nbuf):
          pltpu.make_async_copy(tbl_s.at[src_v.at[0]], bufs[b], sem).wait()
        for b in range(nbuf):
          pltpu.sync_copy(bufs[b], acc.at[dst_v.at[jo + b]], add=True)
        for b in range(nbuf):
          jn = jo + b + nbuf
          jn = lax.select(jn >= CPW, jn - CPW, jn)
          pltpu.async_copy(tbl_s.at[src_v.at[jn]], bufs[b], sem)
      for b in range(nbuf):
        pltpu.make_async_copy(tbl_s.at[src_v.at[0]], bufs[b], sem).wait()
    else:
      @pl.loop(0, CPW)
      def _(j):
        pltpu.async_copy(rows_v, acc.at[dst_v.at[j]], sem, add=True)

      @pl.loop(0, CPW)
      def _(j):
        pltpu.make_async_copy(rows_v, acc.at[dst_v.at[0]], sem).wait()

    plsc.subcore_barrier()
    pltpu.sync_copy(acc.at[pl.ds(s * ROWS_PT, ROWS_PT)],
                    out_h.at[c, pl.ds(s * ROWS_PT, ROWS_PT)])

  return pl.kernel(
      body,
      out_type=jax.ShapeDtypeStruct((NC, N_PAD, F_MID), jnp.float32),
      mesh=_mesh,
      scratch_types=scratch,
      compiler_params=pltpu.CompilerParams(use_tc_tiling_on_sc=False),
  )


_sc_deg = _make_sc_pass(gather=False)
_sc_agg = _make_sc_pass(gather=True)

_RB = 2000
_GRID = N // _RB


def _tc1(x, w1, degp):
  def body(x_ref, w_ref, p_ref, hs_ref, dv_ref):
    deg = p_ref[0] + p_ref[1] + 1.0
    dinv = lax.rsqrt(deg)
    h = jnp.dot(x_ref[...], w_ref[...], preferred_element_type=jnp.float32)
    hs_ref[...] = dinv * h
    dv_ref[...] = dinv

  return pl.pallas_call(
      body,
      grid=(_GRID,),
      in_specs=[
          pl.BlockSpec((_RB, F_IN), lambda i: (i, 0)),
          pl.BlockSpec((F_IN, F_MID), lambda i: (0, 0)),
          pl.BlockSpec((NC, _RB, F_MID), lambda i: (0, i, 0)),
      ],
      out_specs=[pl.BlockSpec((_RB, F_MID), lambda i: (i, 0))] * 2,
      out_shape=[jax.ShapeDtypeStruct((N, F_MID), jnp.float32)] * 2,
  )(x, w1, degp)


def _tc2(p, hs1, dinv, b1b):
  def body(p_ref, hs_ref, dv_ref, b_ref, o_ref):
    z = dv_ref[...] * (p_ref[0] + p_ref[1] + hs_ref[...]) + b_ref[0:1, :]
    o_ref[...] = dv_ref[...] * jnp.maximum(z, 0.0)

  return pl.pallas_call(
      body,
      grid=(_GRID,),
      in_specs=[
          pl.BlockSpec((NC, _RB, F_MID), lambda i: (0, i, 0)),
          pl.BlockSpec((_RB, F_MID), lambda i: (i, 0)),
          pl.BlockSpec((_RB, F_MID), lambda i: (i, 0)),
          pl.BlockSpec((8, F_MID), lambda i: (0, 0)),
      ],
      out_specs=pl.BlockSpec((_RB, F_MID), lambda i: (i, 0)),
      out_shape=jax.ShapeDtypeStruct((N, F_MID), jnp.float32),
  )(p, hs1, dinv, b1b)


def _tc3(p, hs2, dinv, w2, b2b, wlin, blinb):
  def body(p_ref, hs_ref, dv_ref, w2_ref, b2_ref, wl_ref, bl_ref, o_ref):
    agg = dv_ref[...] * (p_ref[0] + p_ref[1] + hs_ref[...])
    h2 = jnp.dot(agg, w2_ref[...], preferred_element_type=jnp.float32)
    h2 = jnp.maximum(h2 + b2_ref[0:1, :], 0.0)
    o_ref[...] = (jnp.dot(h2, wl_ref[...], preferred_element_type=jnp.float32)
                  + bl_ref[0:1, 0:1])

  return pl.pallas_call(
      body,
      grid=(_GRID,),
      in_specs=[
          pl.BlockSpec((NC, _RB, F_MID), lambda i: (0, i, 0)),
          pl.BlockSpec((_RB, F_MID), lambda i: (i, 0)),
          pl.BlockSpec((_RB, F_MID), lambda i: (i, 0)),
          pl.BlockSpec((F_MID, F_IN), lambda i: (0, 0)),
          pl.BlockSpec((8, F_IN), lambda i: (0, 0)),
          pl.BlockSpec((F_IN, 1), lambda i: (0, 0)),
          pl.BlockSpec((8, 8), lambda i: (0, 0)),
      ],
      out_specs=pl.BlockSpec((_RB, 1), lambda i: (i, 0)),
      out_shape=jax.ShapeDtypeStruct((N, 1), jnp.float32),
  )(p, hs2, dinv, w2, b2b, wlin, blinb)


@jax.jit
def kernel(x, edge_index, W1, b1, W2, b2, Wlin, blin):
  src = edge_index[0].astype(jnp.int32)
  dst = edge_index[1].astype(jnp.int32)
  pad = E_PAD - E
  src2d = jnp.concatenate(
      [src, jnp.zeros((pad,), jnp.int32)]).reshape(NW, CPW, CHUNK)
  dst2d = jnp.concatenate(
      [dst, jnp.full((pad,), N_PAD - 1, jnp.int32)]).reshape(NW, CPW, CHUNK)
  ones_rows = jnp.ones((CHUNK, F_MID), jnp.float32)
  zrows = jnp.zeros((ROWS_PT, F_MID), jnp.float32)
  b1b = jnp.broadcast_to(b1.reshape(1, F_MID), (8, F_MID))
  b2b = jnp.broadcast_to(b2.reshape(1, F_IN), (8, F_IN))
  blinb = jnp.broadcast_to(blin.reshape(1, 1), (8, 8))

  degp = _sc_deg(ones_rows, dst2d, zrows)
  hs1, dinv = _tc1(x, W1, degp)
  hs1p = jnp.pad(hs1, ((0, N_PAD - N), (0, 0)))
  p1 = _sc_agg(hs1p, src2d, dst2d, zrows)
  hs2 = _tc2(p1, hs1, dinv, b1b)
  hs2p = jnp.pad(hs2, ((0, N_PAD - N), (0, 0)))
  p2 = _sc_agg(hs2p, src2d, dst2d, zrows)
  return _tc3(p2, hs2, dinv, W2, b2b, Wlin, blinb)


# speedup vs baseline: 57.4398x; 1.1303x over previous
import jax
import jax.numpy as jnp
from jax import lax
from jax.experimental import pallas as pl
from jax.experimental.pallas import tpu as pltpu
from jax.experimental.pallas import tpu_sc as plsc

N = 10000
E = 320000
F_IN = 128
F_MID = 16

NC = 2
NS = 16
NW = NC * NS
CHUNK = 512
CPW = 20
NBUF = 4
E_PAD = NW * CPW * CHUNK
N_PAD = 10240
ROWS_PT = N_PAD // NS

_mesh = plsc.VectorSubcoreMesh(core_axis_name="c", subcore_axis_name="s")
_sc_params = pltpu.CompilerParams(use_tc_tiling_on_sc=False,
                                 needs_layout_passes=False)


def _rsqrt16(d):
  yi = jnp.int32(0x5F3759DF) - (plsc.bitcast(d, jnp.int32) >> 1)
  y = plsc.bitcast(yi, jnp.float32)
  for _ in range(3):
    y = y * (1.5 - 0.5 * d * y * y)
  return y


def _edge_loop(tbl_s, acc, src_v, dst_v, bufs, sem):
  for b in range(NBUF):
    pltpu.async_copy(tbl_s.at[src_v.at[b]], bufs[b], sem)

  @pl.loop(0, CPW, step=NBUF)
  def _(jo):
    for b in range(NBUF):
      pltpu.make_async_copy(tbl_s.at[src_v.at[0]], bufs[b], sem).wait()
    for b in range(NBUF):
      pltpu.sync_copy(bufs[b], acc.at[dst_v.at[jo + b]], add=True)
    for b in range(NBUF):
      jn = jo + b + NBUF
      jn = lax.select(jn >= CPW, jn - CPW, jn)
      pltpu.async_copy(tbl_s.at[src_v.at[jn]], bufs[b], sem)

  for b in range(NBUF):
    pltpu.make_async_copy(tbl_s.at[src_v.at[0]], bufs[b], sem).wait()


def _sc_deg_kernel():
  def body(ones_h, dst_h, zrows_h, out_h, dst_v, ones_v, sem, acc):
    c = lax.axis_index("c")
    s = lax.axis_index("s")
    wid = s * NC + c
    pltpu.sync_copy(zrows_h, acc.at[pl.ds(s * ROWS_PT, ROWS_PT)])
    pltpu.sync_copy(dst_h.at[wid], dst_v)
    pltpu.sync_copy(ones_h, ones_v)
    plsc.subcore_barrier()

    @pl.loop(0, CPW)
    def _(j):
      pltpu.async_copy(ones_v, acc.at[dst_v.at[j]], sem, add=True)

    @pl.loop(0, CPW)
    def _(j):
      pltpu.make_async_copy(ones_v, acc.at[dst_v.at[0]], sem).wait()

    plsc.subcore_barrier()
    pltpu.sync_copy(acc.at[pl.ds(s * ROWS_PT, ROWS_PT)],
                    out_h.at[c, pl.ds(s * ROWS_PT, ROWS_PT)])

  return pl.kernel(
      body,
      out_type=jax.ShapeDtypeStruct((NC, N_PAD, F_MID), jnp.float32),
      mesh=_mesh,
      scratch_types=[
          pltpu.VMEM((CPW, CHUNK), jnp.int32),
          pltpu.VMEM((CHUNK, F_MID), jnp.float32),
          pltpu.SemaphoreType.DMA,
          pltpu.VMEM_SHARED((N_PAD, F_MID), jnp.float32),
      ],
      compiler_params=_sc_params,
  )


def _agg_scratch(extra):
  return [
      pltpu.VMEM((CPW, CHUNK), jnp.int32),
      pltpu.VMEM((CPW, CHUNK), jnp.int32),
      pltpu.VMEM((ROWS_PT, F_MID), jnp.float32),
      pltpu.VMEM((ROWS_PT, F_MID), jnp.float32),
      pltpu.VMEM((ROWS_PT, F_MID), jnp.float32),
      pltpu.VMEM((ROWS_PT, F_MID), jnp.float32),
      pltpu.SemaphoreType.DMA,
      pltpu.VMEM_SHARED((N_PAD, F_MID), jnp.float32),
      pltpu.VMEM_SHARED((N_PAD, F_MID), jnp.float32),
  ] + extra + [pltpu.VMEM((CHUNK, F_MID), jnp.float32) for _ in range(NBUF)]


def _sc_agg1_kernel():
  def body(h1_h, degp_h, src_h, dst_h, zrows_h,
           p_h, hs_out_h, dv_out_h,
           src_v, dst_v, t0, t1, th, tdv, sem, acc, tbl_s, *bufs):
    c = lax.axis_index("c")
    s = lax.axis_index("s")
    wid = s * NC + c
    rsl = pl.ds(s * ROWS_PT, ROWS_PT)
    pltpu.sync_copy(zrows_h, acc.at[rsl])
    pltpu.sync_copy(src_h.at[wid], src_v)
    pltpu.sync_copy(dst_h.at[wid], dst_v)
    pltpu.sync_copy(degp_h.at[0, rsl], t0)
    pltpu.sync_copy(degp_h.at[1, rsl], t1)
    pltpu.sync_copy(h1_h.at[rsl], th)

    @pl.loop(0, ROWS_PT)
    def _(i):
      dinv = _rsqrt16(t0[i, :] + t1[i, :] + 1.0)
      tdv[i, :] = dinv
      th[i, :] = dinv * th[i, :]

    pltpu.sync_copy(th, tbl_s.at[rsl])

    @pl.when(c == 0)
    def _():
      pltpu.sync_copy(th, hs_out_h.at[rsl])
      pltpu.sync_copy(tdv, dv_out_h.at[rsl])

    plsc.subcore_barrier()
    _edge_loop(tbl_s, acc, src_v, dst_v, bufs, sem)
    plsc.subcore_barrier()
    pltpu.sync_copy(acc.at[rsl], p_h.at[c, rsl])

  return pl.kernel(
      body,
      out_type=(jax.ShapeDtypeStruct((NC, N_PAD, F_MID), jnp.float32),
                jax.ShapeDtypeStruct((N_PAD, F_MID), jnp.float32),
                jax.ShapeDtypeStruct((N_PAD, F_MID), jnp.float32)),
      mesh=_mesh,
      scratch_types=_agg_scratch([]),
      compiler_params=_sc_params,
  )


def _sc_agg2_kernel():
  def body(p1_h, hs1_h, dv_h, b1_h, src_h, dst_h, zrows_h,
           p_h, hs_out_h,
           src_v, dst_v, t0, t1, th, tdv, sem, acc, tbl_s, tb, *bufs):
    c = lax.axis_index("c")
    s = lax.axis_index("s")
    wid = s * NC + c
    rsl = pl.ds(s * ROWS_PT, ROWS_PT)
    pltpu.sync_copy(zrows_h, acc.at[rsl])
    pltpu.sync_copy(src_h.at[wid], src_v)
    pltpu.sync_copy(dst_h.at[wid], dst_v)
    pltpu.sync_copy(p1_h.at[0, rsl], t0)
    pltpu.sync_copy(p1_h.at[1, rsl], t1)
    pltpu.sync_copy(hs1_h.at[rsl], th)
    pltpu.sync_copy(dv_h.at[rsl], tdv)
    pltpu.sync_copy(b1_h, tb)

    @pl.loop(0, ROWS_PT)
    def _(i):
      dinv = tdv[i, :]
      z = dinv * (t0[i, :] + t1[i, :] + th[i, :]) + tb[:]
      th[i, :] = dinv * jnp.maximum(z, 0.0)

    pltpu.sync_copy(th, tbl_s.at[rsl])

    @pl.when(c == 0)
    def _():
      pltpu.sync_copy(th, hs_out_h.at[rsl])

    plsc.subcore_barrier()
    _edge_loop(tbl_s, acc, src_v, dst_v, bufs, sem)
    plsc.subcore_barrier()
    pltpu.sync_copy(acc.at[rsl], p_h.at[c, rsl])

  return pl.kernel(
      body,
      out_type=(jax.ShapeDtypeStruct((NC, N_PAD, F_MID), jnp.float32),
                jax.ShapeDtypeStruct((N_PAD, F_MID), jnp.float32)),
      mesh=_mesh,
      scratch_types=_agg_scratch([pltpu.VMEM((F_MID,), jnp.float32)]),
      compiler_params=_sc_params,
  )


_sc_deg = _sc_deg_kernel()
_sc_agg1 = _sc_agg1_kernel()
_sc_agg2 = _sc_agg2_kernel()

_RB = 2000
_GRID = N // _RB


def _tc_a(x, w1):
  def body(x_ref, w_ref, o_ref):
    o_ref[...] = jnp.dot(x_ref[...], w_ref[...],
                         preferred_element_type=jnp.float32)

  return pl.pallas_call(
      body,
      grid=(_GRID,),
      in_specs=[
          pl.BlockSpec((_RB, F_IN), lambda i: (i, 0)),
          pl.BlockSpec((F_IN, F_MID), lambda i: (0, 0)),
      ],
      out_specs=pl.BlockSpec((_RB, F_MID), lambda i: (i, 0)),
      out_shape=jax.ShapeDtypeStruct((N_PAD, F_MID), jnp.float32),
  )(x, w1)


def _tc_b(p, hs2, dinv, w2, b2b, wlin, blinb):
  def body(p_ref, hs_ref, dv_ref, w2_ref, b2_ref, wl_ref, bl_ref, o_ref):
    agg = dv_ref[...] * (p_ref[0] + p_ref[1] + hs_ref[...])
    h2 = jnp.dot(agg, w2_ref[...], preferred_element_type=jnp.float32)
    h2 = jnp.maximum(h2 + b2_ref[0:1, :], 0.0)
    o_ref[...] = (jnp.dot(h2, wl_ref[...], preferred_element_type=jnp.float32)
                  + bl_ref[0:1, 0:1])

  return pl.pallas_call(
      body,
      grid=(_GRID,),
      in_specs=[
          pl.BlockSpec((NC, _RB, F_MID), lambda i: (0, i, 0)),
          pl.BlockSpec((_RB, F_MID), lambda i: (i, 0)),
          pl.BlockSpec((_RB, F_MID), lambda i: (i, 0)),
          pl.BlockSpec((F_MID, F_IN), lambda i: (0, 0)),
          pl.BlockSpec((8, F_IN), lambda i: (0, 0)),
          pl.BlockSpec((F_IN, 1), lambda i: (0, 0)),
          pl.BlockSpec((8, 8), lambda i: (0, 0)),
      ],
      out_specs=pl.BlockSpec((_RB, 1), lambda i: (i, 0)),
      out_shape=jax.ShapeDtypeStruct((N, 1), jnp.float32),
  )(p, hs2, dinv, w2, b2b, wlin, blinb)


@jax.jit
def kernel(x, edge_index, W1, b1, W2, b2, Wlin, blin):
  src = edge_index[0].astype(jnp.int32)
  dst = edge_index[1].astype(jnp.int32)
  pad = E_PAD - E
  src3 = jnp.concatenate(
      [src, jnp.zeros((pad,), jnp.int32)]).reshape(NW, CPW, CHUNK)
  dst3 = jnp.concatenate(
      [dst, jnp.full((pad,), N_PAD - 1, jnp.int32)]).reshape(NW, CPW, CHUNK)
  ones_rows = jnp.ones((CHUNK, F_MID), jnp.float32)
  zrows = jnp.zeros((ROWS_PT, F_MID), jnp.float32)
  b2b = jnp.broadcast_to(b2.reshape(1, F_IN), (8, F_IN))
  blinb = jnp.broadcast_to(blin.reshape(1, 1), (8, 8))

  h1 = _tc_a(x, W1)
  degp = _sc_deg(ones_rows, dst3, zrows)
  p1, hs1, dinv = _sc_agg1(h1, degp, src3, dst3, zrows)
  p2, hs2 = _sc_agg2(p1, hs1, dinv, b1, src3, dst3, zrows)
  return _tc_b(p2, hs2, dinv, W2, b2b, Wlin, blinb)


# speedup vs baseline: 58.7501x; 1.0228x over previous
import jax
import jax.numpy as jnp
from jax import lax
from jax.experimental import pallas as pl
from jax.experimental.pallas import tpu as pltpu
from jax.experimental.pallas import tpu_sc as plsc

N = 10000
E = 320000
F_IN = 128
F_MID = 16

NC = 2
NS = 16
NW = NC * NS
CHUNK = 512
CPW = 20
NBUF = 4
E_PAD = NW * CPW * CHUNK
N_PAD = 10240
ROWS_PT = N_PAD // NS

_mesh = plsc.VectorSubcoreMesh(core_axis_name="c", subcore_axis_name="s")
_sc_params = pltpu.CompilerParams(use_tc_tiling_on_sc=False,
                                 needs_layout_passes=False)


def _rsqrt16(d):
  yi = jnp.int32(0x5F3759DF) - (plsc.bitcast(d, jnp.int32) >> 1)
  y = plsc.bitcast(yi, jnp.float32)
  for _ in range(3):
    y = y * (1.5 - 0.5 * d * y * y)
  return y


def _edge_loop(tbl_s, acc, src_v, dst_v, bufs, semg, sems):
  assert CPW % 2 == 0
  rounds = CPW // 2
  rings = [(0, 1), (2, 3)]

  def gather(j, b):
    pltpu.async_copy(tbl_s.at[src_v.at[j]], bufs[b], semg[b])

  def gwait(b):
    pltpu.make_async_copy(tbl_s.at[src_v.at[0]], bufs[b], semg[b]).wait()

  def scat(j, b):
    pltpu.async_copy(bufs[b], acc.at[dst_v.at[j]], sems[b], add=True)

  def swait(b):
    pltpu.make_async_copy(bufs[b], acc.at[dst_v.at[0]], sems[b]).wait()

  for b in range(NBUF):
    gather(b, b)
  for r in range(rounds):
    ring = rings[r % 2]
    for k in (0, 1):
      gwait(ring[k])
    for k in (0, 1):
      scat(2 * r + k, ring[k])
    if r >= 1:
      prev = rings[(r - 1) % 2]
      for k in (0, 1):
        swait(prev[k])
        if 2 * (r + 1) + k < CPW:
          gather(2 * (r + 1) + k, prev[k])
  for k in (0, 1):
    swait(rings[(rounds - 1) % 2][k])


def _sc_deg_kernel():
  def body(ones_h, dst_h, zrows_h, out_h, dst_v, ones_v, sem, acc):
    c = lax.axis_index("c")
    s = lax.axis_index("s")
    wid = s * NC + c
    pltpu.sync_copy(zrows_h, acc.at[pl.ds(s * ROWS_PT, ROWS_PT)])
    pltpu.sync_copy(dst_h.at[wid], dst_v)
    pltpu.sync_copy(ones_h, ones_v)
    plsc.subcore_barrier()

    @pl.loop(0, CPW)
    def _(j):
      pltpu.async_copy(ones_v, acc.at[dst_v.at[j]], sem, add=True)

    @pl.loop(0, CPW)
    def _(j):
      pltpu.make_async_copy(ones_v, acc.at[dst_v.at[0]], sem).wait()

    plsc.subcore_barrier()
    pltpu.sync_copy(acc.at[pl.ds(s * ROWS_PT, ROWS_PT)],
                    out_h.at[c, pl.ds(s * ROWS_PT, ROWS_PT)])

  return pl.kernel(
      body,
      out_type=jax.ShapeDtypeStruct((NC, N_PAD, F_MID), jnp.float32),
      mesh=_mesh,
      scratch_types=[
          pltpu.VMEM((CPW, CHUNK), jnp.int32),
          pltpu.VMEM((CHUNK, F_MID), jnp.float32),
          pltpu.SemaphoreType.DMA,
          pltpu.VMEM_SHARED((N_PAD, F_MID), jnp.float32),
      ],
      compiler_params=_sc_params,
  )


def _agg_scratch(extra):
  return [
      pltpu.VMEM((CPW, CHUNK), jnp.int32),
      pltpu.VMEM((CPW, CHUNK), jnp.int32),
      pltpu.VMEM((ROWS_PT, F_MID), jnp.float32),
      pltpu.VMEM((ROWS_PT, F_MID), jnp.float32),
      pltpu.VMEM((ROWS_PT, F_MID), jnp.float32),
      pltpu.VMEM((ROWS_PT, F_MID), jnp.float32),
      pltpu.VMEM_SHARED((N_PAD, F_MID), jnp.float32),
      pltpu.VMEM_SHARED((N_PAD, F_MID), jnp.float32),
  ] + extra + [pltpu.VMEM((CHUNK, F_MID), jnp.float32) for _ in range(NBUF)] \
    + [pltpu.SemaphoreType.DMA for _ in range(2 * NBUF)]


def _sc_agg1_kernel():
  def body(h1_h, degp_h, src_h, dst_h, zrows_h,
           p_h, hs_out_h, dv_out_h,
           src_v, dst_v, t0, t1, th, tdv, acc, tbl_s, *rest):
    bufs, semg, sems = rest[:NBUF], rest[NBUF:2 * NBUF], rest[2 * NBUF:]
    c = lax.axis_index("c")
    s = lax.axis_index("s")
    wid = s * NC + c
    rsl = pl.ds(s * ROWS_PT, ROWS_PT)
    pltpu.sync_copy(zrows_h, acc.at[rsl])
    pltpu.sync_copy(src_h.at[wid], src_v)
    pltpu.sync_copy(dst_h.at[wid], dst_v)
    pltpu.sync_copy(degp_h.at[0, rsl], t0)
    pltpu.sync_copy(degp_h.at[1, rsl], t1)
    pltpu.sync_copy(h1_h.at[rsl], th)

    @pl.loop(0, ROWS_PT)
    def _(i):
      dinv = _rsqrt16(t0[i, :] + t1[i, :] + 1.0)
      tdv[i, :] = dinv
      th[i, :] = dinv * th[i, :]

    pltpu.sync_copy(th, tbl_s.at[rsl])

    @pl.when(c == 0)
    def _():
      pltpu.sync_copy(th, hs_out_h.at[rsl])
      pltpu.sync_copy(tdv, dv_out_h.at[rsl])

    plsc.subcore_barrier()
    _edge_loop(tbl_s, acc, src_v, dst_v, bufs, semg, sems)
    plsc.subcore_barrier()
    pltpu.sync_copy(acc.at[rsl], p_h.at[c, rsl])

  return pl.kernel(
      body,
      out_type=(jax.ShapeDtypeStruct((NC, N_PAD, F_MID), jnp.float32),
                jax.ShapeDtypeStruct((N_PAD, F_MID), jnp.float32),
                jax.ShapeDtypeStruct((N_PAD, F_MID), jnp.float32)),
      mesh=_mesh,
      scratch_types=_agg_scratch([]),
      compiler_params=_sc_params,
  )


def _sc_agg2_kernel():
  def body(p1_h, hs1_h, dv_h, b1_h, src_h, dst_h, zrows_h,
           p_h, hs_out_h,
           src_v, dst_v, t0, t1, th, tdv, acc, tbl_s, tb, *rest):
    bufs, semg, sems = rest[:NBUF], rest[NBUF:2 * NBUF], rest[2 * NBUF:]
    c = lax.axis_index("c")
    s = lax.axis_index("s")
    wid = s * NC + c
    rsl = pl.ds(s * ROWS_PT, ROWS_PT)
    pltpu.sync_copy(zrows_h, acc.at[rsl])
    pltpu.sync_copy(src_h.at[wid], src_v)
    pltpu.sync_copy(dst_h.at[wid], dst_v)
    pltpu.sync_copy(p1_h.at[0, rsl], t0)
    pltpu.sync_copy(p1_h.at[1, rsl], t1)
    pltpu.sync_copy(hs1_h.at[rsl], th)
    pltpu.sync_copy(dv_h.at[rsl], tdv)
    pltpu.sync_copy(b1_h, tb)

    @pl.loop(0, ROWS_PT)
    def _(i):
      dinv = tdv[i, :]
      z = dinv * (t0[i, :] + t1[i, :] + th[i, :]) + tb[:]
      th[i, :] = dinv * jnp.maximum(z, 0.0)

    pltpu.sync_copy(th, tbl_s.at[rsl])

    @pl.when(c == 0)
    def _():
      pltpu.sync_copy(th, hs_out_h.at[rsl])

    plsc.subcore_barrier()
    _edge_loop(tbl_s, acc, src_v, dst_v, bufs, semg, sems)
    plsc.subcore_barrier()
    pltpu.sync_copy(acc.at[rsl], p_h.at[c, rsl])

  return pl.kernel(
      body,
      out_type=(jax.ShapeDtypeStruct((NC, N_PAD, F_MID), jnp.float32),
                jax.ShapeDtypeStruct((N_PAD, F_MID), jnp.float32)),
      mesh=_mesh,
      scratch_types=_agg_scratch([pltpu.VMEM((F_MID,), jnp.float32)]),
      compiler_params=_sc_params,
  )


_sc_deg = _sc_deg_kernel()
_sc_agg1 = _sc_agg1_kernel()
_sc_agg2 = _sc_agg2_kernel()

_RB = 2000
_GRID = N // _RB


def _tc_a(x, w1):
  def body(x_ref, w_ref, o_ref):
    o_ref[...] = jnp.dot(x_ref[...], w_ref[...],
                         preferred_element_type=jnp.float32)

  return pl.pallas_call(
      body,
      grid=(_GRID,),
      in_specs=[
          pl.BlockSpec((_RB, F_IN), lambda i: (i, 0)),
          pl.BlockSpec((F_IN, F_MID), lambda i: (0, 0)),
      ],
      out_specs=pl.BlockSpec((_RB, F_MID), lambda i: (i, 0)),
      out_shape=jax.ShapeDtypeStruct((N_PAD, F_MID), jnp.float32),
  )(x, w1)


def _tc_b(p, hs2, dinv, w2, b2b, wlin, blinb):
  def body(p_ref, hs_ref, dv_ref, w2_ref, b2_ref, wl_ref, bl_ref, o_ref):
    agg = dv_ref[...] * (p_ref[0] + p_ref[1] + hs_ref[...])
    h2 = jnp.dot(agg, w2_ref[...], preferred_element_type=jnp.float32)
    h2 = jnp.maximum(h2 + b2_ref[0:1, :], 0.0)
    o_ref[...] = (jnp.dot(h2, wl_ref[...], preferred_element_type=jnp.float32)
                  + bl_ref[0:1, 0:1])

  return pl.pallas_call(
      body,
      grid=(_GRID,),
      in_specs=[
          pl.BlockSpec((NC, _RB, F_MID), lambda i: (0, i, 0)),
          pl.BlockSpec((_RB, F_MID), lambda i: (i, 0)),
          pl.BlockSpec((_RB, F_MID), lambda i: (i, 0)),
          pl.BlockSpec((F_MID, F_IN), lambda i: (0, 0)),
          pl.BlockSpec((8, F_IN), lambda i: (0, 0)),
          pl.BlockSpec((F_IN, 1), lambda i: (0, 0)),
          pl.BlockSpec((8, 8), lambda i: (0, 0)),
      ],
      out_specs=pl.BlockSpec((_RB, 1), lambda i: (i, 0)),
      out_shape=jax.ShapeDtypeStruct((N, 1), jnp.float32),
  )(p, hs2, dinv, w2, b2b, wlin, blinb)


@jax.jit
def kernel(x, edge_index, W1, b1, W2, b2, Wlin, blin):
  src = edge_index[0].astype(jnp.int32)
  dst = edge_index[1].astype(jnp.int32)
  pad = E_PAD - E
  src3 = jnp.concatenate(
      [src, jnp.zeros((pad,), jnp.int32)]).reshape(NW, CPW, CHUNK)
  dst3 = jnp.concatenate(
      [dst, jnp.full((pad,), N_PAD - 1, jnp.int32)]).reshape(NW, CPW, CHUNK)
  ones_rows = jnp.ones((CHUNK, F_MID), jnp.float32)
  zrows = jnp.zeros((ROWS_PT, F_MID), jnp.float32)
  b2b = jnp.broadcast_to(b2.reshape(1, F_IN), (8, F_IN))
  blinb = jnp.broadcast_to(blin.reshape(1, 1), (8, 8))

  h1 = _tc_a(x, W1)
  degp = _sc_deg(ones_rows, dst3, zrows)
  p1, hs1, dinv = _sc_agg1(h1, degp, src3, dst3, zrows)
  p2, hs2 = _sc_agg2(p1, hs1, dinv, b1, src3, dst3, zrows)
  return _tc_b(p2, hs2, dinv, W2, b2b, Wlin, blinb)


# speedup vs baseline: 60.2182x; 1.0250x over previous
import jax
import jax.numpy as jnp
from jax import lax
from jax.experimental import pallas as pl
from jax.experimental.pallas import tpu as pltpu
from jax.experimental.pallas import tpu_sc as plsc

N = 10000
E = 320000
F_IN = 128
F_MID = 16

NC = 2
NS = 16
NW = NC * NS
CHUNK = 512
CPW = 20
NBUF = 4
E_PAD = NW * CPW * CHUNK
N_PAD = 10240
ROWS_PT = N_PAD // NS

_mesh = plsc.VectorSubcoreMesh(core_axis_name="c", subcore_axis_name="s")
_sc_params = pltpu.CompilerParams(use_tc_tiling_on_sc=False,
                                 needs_layout_passes=False)


def _rsqrt16(d):
  yi = jnp.int32(0x5F3759DF) - (plsc.bitcast(d, jnp.int32) >> 1)
  y = plsc.bitcast(yi, jnp.float32)
  for _ in range(3):
    y = y * (1.5 - 0.5 * d * y * y)
  return y


def _edge_loop(tbl_s, acc, src_v, dst_v, bufs, semg, sems):
  assert CPW % 2 == 0
  rounds = CPW // 2
  rings = [(0, 1), (2, 3)]

  def gather(j, b):
    pltpu.async_copy(tbl_s.at[src_v.at[j]], bufs[b], semg[b])

  def gwait(b):
    pltpu.make_async_copy(tbl_s.at[src_v.at[0]], bufs[b], semg[b]).wait()

  def scat(j, b):
    pltpu.async_copy(bufs[b], acc.at[dst_v.at[j]], sems[b], add=True)

  def swait(b):
    pltpu.make_async_copy(bufs[b], acc.at[dst_v.at[0]], sems[b]).wait()

  for b in range(NBUF):
    gather(b, b)
  for r in range(rounds):
    ring = rings[r % 2]
    for k in (0, 1):
      gwait(ring[k])
    for k in (0, 1):
      scat(2 * r + k, ring[k])
    if r >= 1:
      prev = rings[(r - 1) % 2]
      for k in (0, 1):
        swait(prev[k])
        if 2 * (r + 1) + k < CPW:
          gather(2 * (r + 1) + k, prev[k])
  for k in (0, 1):
    swait(rings[(rounds - 1) % 2][k])


def _sc_deg_kernel():
  def body(ones_h, dst_h, zrows_h, out_h, dst_v, ones_v, sem, acc):
    c = lax.axis_index("c")
    s = lax.axis_index("s")
    wid = s * NC + c
    pltpu.sync_copy(zrows_h, acc.at[pl.ds(s * ROWS_PT, ROWS_PT)])
    pltpu.sync_copy(dst_h.at[wid], dst_v)
    pltpu.sync_copy(ones_h, ones_v)
    plsc.subcore_barrier()

    @pl.loop(0, CPW)
    def _(j):
      pltpu.async_copy(ones_v, acc.at[dst_v.at[j]], sem, add=True)

    @pl.loop(0, CPW)
    def _(j):
      pltpu.make_async_copy(ones_v, acc.at[dst_v.at[0]], sem).wait()

    plsc.subcore_barrier()
    pltpu.sync_copy(acc.at[pl.ds(s * ROWS_PT, ROWS_PT)],
                    out_h.at[c, pl.ds(s * ROWS_PT, ROWS_PT)])

  return pl.kernel(
      body,
      out_type=jax.ShapeDtypeStruct((NC, N_PAD, F_MID), jnp.float32),
      mesh=_mesh,
      scratch_types=[
          pltpu.VMEM((CPW, CHUNK), jnp.int32),
          pltpu.VMEM((CHUNK, F_MID), jnp.float32),
          pltpu.SemaphoreType.DMA,
          pltpu.VMEM_SHARED((N_PAD, F_MID), jnp.float32),
      ],
      compiler_params=_sc_params,
  )


def _agg_scratch(extra):
  return [
      pltpu.VMEM((CPW, CHUNK), jnp.int32),
      pltpu.VMEM((CPW, CHUNK), jnp.int32),
      pltpu.VMEM((ROWS_PT, F_MID), jnp.float32),
      pltpu.VMEM((ROWS_PT, F_MID), jnp.float32),
      pltpu.VMEM((ROWS_PT, F_MID), jnp.float32),
      pltpu.VMEM((ROWS_PT, F_MID), jnp.float32),
      pltpu.VMEM_SHARED((N_PAD, F_MID), jnp.float32),
      pltpu.VMEM_SHARED((N_PAD, F_MID), jnp.float32),
  ] + extra + [pltpu.VMEM((CHUNK, F_MID), jnp.float32) for _ in range(NBUF)] \
    + [pltpu.SemaphoreType.DMA for _ in range(2 * NBUF)]


def _sc_agg1_kernel():
  def body(h1_h, degp_h, src_h, dst_h, zrows_h,
           p_h, hs_out_h, dv_out_h,
           src_v, dst_v, t0, t1, th, tdv, acc, tbl_s, *rest):
    bufs, semg, sems = rest[:NBUF], rest[NBUF:2 * NBUF], rest[2 * NBUF:]
    c = lax.axis_index("c")
    s = lax.axis_index("s")
    wid = s * NC + c
    rsl = pl.ds(s * ROWS_PT, ROWS_PT)
    pltpu.sync_copy(zrows_h, acc.at[rsl])
    pltpu.sync_copy(src_h.at[wid], src_v)
    pltpu.sync_copy(dst_h.at[wid], dst_v)
    pltpu.sync_copy(degp_h.at[0, rsl], t0)
    pltpu.sync_copy(degp_h.at[1, rsl], t1)
    pltpu.sync_copy(h1_h.at[rsl], th)

    @plsc.parallel_loop(0, ROWS_PT, unroll=8)
    def _(i):
      dinv = _rsqrt16(t0[i, :] + t1[i, :] + 1.0)
      tdv[i, :] = dinv
      th[i, :] = dinv * th[i, :]

    pltpu.sync_copy(th, tbl_s.at[rsl])

    @pl.when(c == 0)
    def _():
      pltpu.sync_copy(th, hs_out_h.at[rsl])
      pltpu.sync_copy(tdv, dv_out_h.at[rsl])

    plsc.subcore_barrier()
    _edge_loop(tbl_s, acc, src_v, dst_v, bufs, semg, sems)
    plsc.subcore_barrier()
    pltpu.sync_copy(acc.at[rsl], p_h.at[c, rsl])

  return pl.kernel(
      body,
      out_type=(jax.ShapeDtypeStruct((NC, N_PAD, F_MID), jnp.float32),
                jax.ShapeDtypeStruct((N_PAD, F_MID), jnp.float32),
                jax.ShapeDtypeStruct((N_PAD, F_MID), jnp.float32)),
      mesh=_mesh,
      scratch_types=_agg_scratch([]),
      compiler_params=_sc_params,
  )


def _sc_agg2_kernel():
  def body(p1_h, hs1_h, dv_h, b1_h, src_h, dst_h, zrows_h,
           p_h, hs_out_h,
           src_v, dst_v, t0, t1, th, tdv, acc, tbl_s, tb, *rest):
    bufs, semg, sems = rest[:NBUF], rest[NBUF:2 * NBUF], rest[2 * NBUF:]
    c = lax.axis_index("c")
    s = lax.axis_index("s")
    wid = s * NC + c
    rsl = pl.ds(s * ROWS_PT, ROWS_PT)
    pltpu.sync_copy(zrows_h, acc.at[rsl])
    pltpu.sync_copy(src_h.at[wid], src_v)
    pltpu.sync_copy(dst_h.at[wid], dst_v)
    pltpu.sync_copy(p1_h.at[0, rsl], t0)
    pltpu.sync_copy(p1_h.at[1, rsl], t1)
    pltpu.sync_copy(hs1_h.at[rsl], th)
    pltpu.sync_copy(dv_h.at[rsl], tdv)
    pltpu.sync_copy(b1_h, tb)

    @plsc.parallel_loop(0, ROWS_PT, unroll=8)
    def _(i):
      dinv = tdv[i, :]
      z = dinv * (t0[i, :] + t1[i, :] + th[i, :]) + tb[:]
      th[i, :] = dinv * jnp.maximum(z, 0.0)

    pltpu.sync_copy(th, tbl_s.at[rsl])

    @pl.when(c == 0)
    def _():
      pltpu.sync_copy(th, hs_out_h.at[rsl])

    plsc.subcore_barrier()
    _edge_loop(tbl_s, acc, src_v, dst_v, bufs, semg, sems)
    plsc.subcore_barrier()
    pltpu.sync_copy(acc.at[rsl], p_h.at[c, rsl])

  return pl.kernel(
      body,
      out_type=(jax.ShapeDtypeStruct((NC, N_PAD, F_MID), jnp.float32),
                jax.ShapeDtypeStruct((N_PAD, F_MID), jnp.float32)),
      mesh=_mesh,
      scratch_types=_agg_scratch([pltpu.VMEM((F_MID,), jnp.float32)]),
      compiler_params=_sc_params,
  )


_sc_deg = _sc_deg_kernel()
_sc_agg1 = _sc_agg1_kernel()
_sc_agg2 = _sc_agg2_kernel()

_RB = 2000
_GRID = N // _RB


def _tc_a(x, w1):
  def body(x_ref, w_ref, o_ref):
    o_ref[...] = jnp.dot(x_ref[...], w_ref[...],
                         preferred_element_type=jnp.float32)

  return pl.pallas_call(
      body,
      grid=(_GRID,),
      in_specs=[
          pl.BlockSpec((_RB, F_IN), lambda i: (i, 0)),
          pl.BlockSpec((F_IN, F_MID), lambda i: (0, 0)),
      ],
      out_specs=pl.BlockSpec((_RB, F_MID), lambda i: (i, 0)),
      out_shape=jax.ShapeDtypeStruct((N_PAD, F_MID), jnp.float32),
  )(x, w1)


def _tc_b(p, hs2, dinv, w2, b2b, wlin, blinb):
  def body(p_ref, hs_ref, dv_ref, w2_ref, b2_ref, wl_ref, bl_ref, o_ref):
    agg = dv_ref[...] * (p_ref[0] + p_ref[1] + hs_ref[...])
    h2 = jnp.dot(agg, w2_ref[...], preferred_element_type=jnp.float32)
    h2 = jnp.maximum(h2 + b2_ref[0:1, :], 0.0)
    o_ref[...] = (jnp.dot(h2, wl_ref[...], preferred_element_type=jnp.float32)
                  + bl_ref[0:1, 0:1])

  return pl.pallas_call(
      body,
      grid=(_GRID,),
      in_specs=[
          pl.BlockSpec((NC, _RB, F_MID), lambda i: (0, i, 0)),
          pl.BlockSpec((_RB, F_MID), lambda i: (i, 0)),
          pl.BlockSpec((_RB, F_MID), lambda i: (i, 0)),
          pl.BlockSpec((F_MID, F_IN), lambda i: (0, 0)),
          pl.BlockSpec((8, F_IN), lambda i: (0, 0)),
          pl.BlockSpec((F_IN, 1), lambda i: (0, 0)),
          pl.BlockSpec((8, 8), lambda i: (0, 0)),
      ],
      out_specs=pl.BlockSpec((_RB, 1), lambda i: (i, 0)),
      out_shape=jax.ShapeDtypeStruct((N, 1), jnp.float32),
  )(p, hs2, dinv, w2, b2b, wlin, blinb)


@jax.jit
def kernel(x, edge_index, W1, b1, W2, b2, Wlin, blin):
  src = edge_index[0].astype(jnp.int32)
  dst = edge_index[1].astype(jnp.int32)
  pad = E_PAD - E
  src3 = jnp.concatenate(
      [src, jnp.zeros((pad,), jnp.int32)]).reshape(NW, CPW, CHUNK)
  dst3 = jnp.concatenate(
      [dst, jnp.full((pad,), N_PAD - 1, jnp.int32)]).reshape(NW, CPW, CHUNK)
  ones_rows = jnp.ones((CHUNK, F_MID), jnp.float32)
  zrows = jnp.zeros((ROWS_PT, F_MID), jnp.float32)
  b2b = jnp.broadcast_to(b2.reshape(1, F_IN), (8, F_IN))
  blinb = jnp.broadcast_to(blin.reshape(1, 1), (8, 8))

  h1 = _tc_a(x, W1)
  degp = _sc_deg(ones_rows, dst3, zrows)
  p1, hs1, dinv = _sc_agg1(h1, degp, src3, dst3, zrows)
  p2, hs2 = _sc_agg2(p1, hs1, dinv, b1, src3, dst3, zrows)
  return _tc_b(p2, hs2, dinv, W2, b2b, Wlin, blinb)


# speedup vs baseline: 63.3010x; 1.0512x over previous
import jax
import jax.numpy as jnp
from jax import lax
from jax.experimental import pallas as pl
from jax.experimental.pallas import tpu as pltpu
from jax.experimental.pallas import tpu_sc as plsc

N = 10000
E = 320000
F_IN = 128
F_MID = 16

NC = 2
NS = 16
NW = NC * NS
CHUNK = 512
CPW = 20
NBUF = 4
E_PAD = NW * CPW * CHUNK
N_PAD = 10240
ROWS_PT = N_PAD // NS

_mesh = plsc.VectorSubcoreMesh(core_axis_name="c", subcore_axis_name="s")
_sc_params = pltpu.CompilerParams(use_tc_tiling_on_sc=False,
                                 needs_layout_passes=False)


def _rsqrt16(d):
  yi = jnp.int32(0x5F3759DF) - (plsc.bitcast(d, jnp.int32) >> 1)
  y = plsc.bitcast(yi, jnp.float32)
  for _ in range(3):
    y = y * (1.5 - 0.5 * d * y * y)
  return y


def _edge_loop(tbl_s, acc, src_v, dst_v, bufs, semg, sems):
  assert CPW % 2 == 0
  rounds = CPW // 2
  rings = [(0, 1), (2, 3)]

  def gather(j, b):
    pltpu.async_copy(tbl_s.at[src_v.at[j]], bufs[b], semg[b])

  def gwait(b):
    pltpu.make_async_copy(tbl_s.at[src_v.at[0]], bufs[b], semg[b]).wait()

  def scat(j, b):
    pltpu.async_copy(bufs[b], acc.at[dst_v.at[j]], sems[b], add=True)

  def swait(b):
    pltpu.make_async_copy(bufs[b], acc.at[dst_v.at[0]], sems[b]).wait()

  for b in range(NBUF):
    gather(b, b)
  for r in range(rounds):
    ring = rings[r % 2]
    for k in (0, 1):
      gwait(ring[k])
    for k in (0, 1):
      scat(2 * r + k, ring[k])
    if r >= 1:
      prev = rings[(r - 1) % 2]
      for k in (0, 1):
        swait(prev[k])
        if 2 * (r + 1) + k < CPW:
          gather(2 * (r + 1) + k, prev[k])
  for k in (0, 1):
    swait(rings[(rounds - 1) % 2][k])


def _sc_deg_kernel():
  def body(ones_h, dst_h, zrows_h, out_h, dst_v, ones_v, sem, acc):
    c = lax.axis_index("c")
    s = lax.axis_index("s")
    wid = s * NC + c
    a0 = pltpu.async_copy(zrows_h, acc.at[pl.ds(s * ROWS_PT, ROWS_PT)], sem)
    a1 = pltpu.async_copy(dst_h.at[wid], dst_v, sem)
    a2 = pltpu.async_copy(ones_h, ones_v, sem)
    a0.wait(); a1.wait(); a2.wait()
    plsc.subcore_barrier()

    @pl.loop(0, CPW)
    def _(j):
      pltpu.async_copy(ones_v, acc.at[dst_v.at[j]], sem, add=True)

    @pl.loop(0, CPW)
    def _(j):
      pltpu.make_async_copy(ones_v, acc.at[dst_v.at[0]], sem).wait()

    plsc.subcore_barrier()
    pltpu.sync_copy(acc.at[pl.ds(s * ROWS_PT, ROWS_PT)],
                    out_h.at[c, pl.ds(s * ROWS_PT, ROWS_PT)])

  return pl.kernel(
      body,
      out_type=jax.ShapeDtypeStruct((NC, N_PAD, F_MID), jnp.float32),
      mesh=_mesh,
      scratch_types=[
          pltpu.VMEM((CPW, CHUNK), jnp.int32),
          pltpu.VMEM((CHUNK, F_MID), jnp.float32),
          pltpu.SemaphoreType.DMA,
          pltpu.VMEM_SHARED((N_PAD, F_MID), jnp.float32),
      ],
      compiler_params=_sc_params,
  )


def _agg_scratch(extra):
  return [
      pltpu.VMEM((CPW, CHUNK), jnp.int32),
      pltpu.VMEM((CPW, CHUNK), jnp.int32),
      pltpu.VMEM((ROWS_PT, F_MID), jnp.float32),
      pltpu.VMEM((ROWS_PT, F_MID), jnp.float32),
      pltpu.VMEM((ROWS_PT, F_MID), jnp.float32),
      pltpu.VMEM((ROWS_PT, F_MID), jnp.float32),
      pltpu.VMEM_SHARED((N_PAD, F_MID), jnp.float32),
      pltpu.VMEM_SHARED((N_PAD, F_MID), jnp.float32),
  ] + extra + [pltpu.VMEM((CHUNK, F_MID), jnp.float32) for _ in range(NBUF)] \
    + [pltpu.SemaphoreType.DMA for _ in range(2 * NBUF)]


def _sc_agg1_kernel():
  def body(h1_h, degp_h, src_h, dst_h, zrows_h,
           p_h, hs_out_h, dv_out_h,
           src_v, dst_v, t0, t1, th, tdv, acc, tbl_s, *rest):
    bufs, semg, sems = rest[:NBUF], rest[NBUF:2 * NBUF], rest[2 * NBUF:]
    c = lax.axis_index("c")
    s = lax.axis_index("s")
    wid = s * NC + c
    rsl = pl.ds(s * ROWS_PT, ROWS_PT)
    a_acc = pltpu.async_copy(zrows_h, acc.at[rsl], semg[0])
    a_src = pltpu.async_copy(src_h.at[wid], src_v, semg[1])
    a_dst = pltpu.async_copy(dst_h.at[wid], dst_v, semg[2])
    a_t0 = pltpu.async_copy(degp_h.at[0, rsl], t0, semg[3])
    a_t1 = pltpu.async_copy(degp_h.at[1, rsl], t1, sems[0])
    a_th = pltpu.async_copy(h1_h.at[rsl], th, sems[1])
    a_t0.wait(); a_t1.wait(); a_th.wait()

    @plsc.parallel_loop(0, ROWS_PT, unroll=8)
    def _(i):
      dinv = _rsqrt16(t0[i, :] + t1[i, :] + 1.0)
      tdv[i, :] = dinv
      th[i, :] = dinv * th[i, :]

    pltpu.sync_copy(th, tbl_s.at[rsl])

    @pl.when(c == 0)
    def _():
      pltpu.sync_copy(th, hs_out_h.at[rsl])
      pltpu.sync_copy(tdv, dv_out_h.at[rsl])

    a_acc.wait(); a_src.wait(); a_dst.wait()
    plsc.subcore_barrier()
    _edge_loop(tbl_s, acc, src_v, dst_v, bufs, semg, sems)
    plsc.subcore_barrier()
    pltpu.sync_copy(acc.at[rsl], p_h.at[c, rsl])

  return pl.kernel(
      body,
      out_type=(jax.ShapeDtypeStruct((NC, N_PAD, F_MID), jnp.float32),
                jax.ShapeDtypeStruct((N_PAD, F_MID), jnp.float32),
                jax.ShapeDtypeStruct((N_PAD, F_MID), jnp.float32)),
      mesh=_mesh,
      scratch_types=_agg_scratch([]),
      compiler_params=_sc_params,
  )


def _sc_agg2_kernel():
  def body(p1_h, hs1_h, dv_h, b1_h, src_h, dst_h, zrows_h,
           p_h, hs_out_h,
           src_v, dst_v, t0, t1, th, tdv, acc, tbl_s, tb, *rest):
    bufs, semg, sems = rest[:NBUF], rest[NBUF:2 * NBUF], rest[2 * NBUF:]
    c = lax.axis_index("c")
    s = lax.axis_index("s")
    wid = s * NC + c
    rsl = pl.ds(s * ROWS_PT, ROWS_PT)
    a_acc = pltpu.async_copy(zrows_h, acc.at[rsl], semg[0])
    a_src = pltpu.async_copy(src_h.at[wid], src_v, semg[1])
    a_dst = pltpu.async_copy(dst_h.at[wid], dst_v, semg[2])
    a_t0 = pltpu.async_copy(p1_h.at[0, rsl], t0, semg[3])
    a_t1 = pltpu.async_copy(p1_h.at[1, rsl], t1, sems[0])
    a_th = pltpu.async_copy(hs1_h.at[rsl], th, sems[1])
    a_dv = pltpu.async_copy(dv_h.at[rsl], tdv, sems[2])
    a_tb = pltpu.async_copy(b1_h, tb, sems[3])
    a_t0.wait(); a_t1.wait(); a_th.wait(); a_dv.wait(); a_tb.wait()

    @plsc.parallel_loop(0, ROWS_PT, unroll=8)
    def _(i):
      dinv = tdv[i, :]
      z = dinv * (t0[i, :] + t1[i, :] + th[i, :]) + tb[:]
      th[i, :] = dinv * jnp.maximum(z, 0.0)

    pltpu.sync_copy(th, tbl_s.at[rsl])

    @pl.when(c == 0)
    def _():
      pltpu.sync_copy(th, hs_out_h.at[rsl])

    a_acc.wait(); a_src.wait(); a_dst.wait()
    plsc.subcore_barrier()
    _edge_loop(tbl_s, acc, src_v, dst_v, bufs, semg, sems)
    plsc.subcore_barrier()
    pltpu.sync_copy(acc.at[rsl], p_h.at[c, rsl])

  return pl.kernel(
      body,
      out_type=(jax.ShapeDtypeStruct((NC, N_PAD, F_MID), jnp.float32),
                jax.ShapeDtypeStruct((N_PAD, F_MID), jnp.float32)),
      mesh=_mesh,
      scratch_types=_agg_scratch([pltpu.VMEM((F_MID,), jnp.float32)]),
      compiler_params=_sc_params,
  )


_sc_deg = _sc_deg_kernel()
_sc_agg1 = _sc_agg1_kernel()
_sc_agg2 = _sc_agg2_kernel()

_RB = 2000
_GRID = N // _RB


def _tc_a(x, w1):
  def body(x_ref, w_ref, o_ref):
    o_ref[...] = jnp.dot(x_ref[...], w_ref[...],
                         preferred_element_type=jnp.float32)

  return pl.pallas_call(
      body,
      grid=(_GRID,),
      in_specs=[
          pl.BlockSpec((_RB, F_IN), lambda i: (i, 0)),
          pl.BlockSpec((F_IN, F_MID), lambda i: (0, 0)),
      ],
      out_specs=pl.BlockSpec((_RB, F_MID), lambda i: (i, 0)),
      out_shape=jax.ShapeDtypeStruct((N_PAD, F_MID), jnp.float32),
  )(x, w1)


def _tc_b(p, hs2, dinv, w2, b2b, wlin, blinb):
  def body(p_ref, hs_ref, dv_ref, w2_ref, b2_ref, wl_ref, bl_ref, o_ref):
    agg = dv_ref[...] * (p_ref[0] + p_ref[1] + hs_ref[...])
    h2 = jnp.dot(agg, w2_ref[...], preferred_element_type=jnp.float32)
    h2 = jnp.maximum(h2 + b2_ref[0:1, :], 0.0)
    o_ref[...] = (jnp.dot(h2, wl_ref[...], preferred_element_type=jnp.float32)
                  + bl_ref[0:1, 0:1])

  return pl.pallas_call(
      body,
      grid=(_GRID,),
      in_specs=[
          pl.BlockSpec((NC, _RB, F_MID), lambda i: (0, i, 0)),
          pl.BlockSpec((_RB, F_MID), lambda i: (i, 0)),
          pl.BlockSpec((_RB, F_MID), lambda i: (i, 0)),
          pl.BlockSpec((F_MID, F_IN), lambda i: (0, 0)),
          pl.BlockSpec((8, F_IN), lambda i: (0, 0)),
          pl.BlockSpec((F_IN, 1), lambda i: (0, 0)),
          pl.BlockSpec((8, 8), lambda i: (0, 0)),
      ],
      out_specs=pl.BlockSpec((_RB, 1), lambda i: (i, 0)),
      out_shape=jax.ShapeDtypeStruct((N, 1), jnp.float32),
  )(p, hs2, dinv, w2, b2b, wlin, blinb)


@jax.jit
def kernel(x, edge_index, W1, b1, W2, b2, Wlin, blin):
  src = edge_index[0].astype(jnp.int32)
  dst = edge_index[1].astype(jnp.int32)
  pad = E_PAD - E
  src3 = jnp.concatenate(
      [src, jnp.zeros((pad,), jnp.int32)]).reshape(NW, CPW, CHUNK)
  dst3 = jnp.concatenate(
      [dst, jnp.full((pad,), N_PAD - 1, jnp.int32)]).reshape(NW, CPW, CHUNK)
  ones_rows = jnp.ones((CHUNK, F_MID), jnp.float32)
  zrows = jnp.zeros((ROWS_PT, F_MID), jnp.float32)
  b2b = jnp.broadcast_to(b2.reshape(1, F_IN), (8, F_IN))
  blinb = jnp.broadcast_to(blin.reshape(1, 1), (8, 8))

  h1 = _tc_a(x, W1)
  degp = _sc_deg(ones_rows, dst3, zrows)
  p1, hs1, dinv = _sc_agg1(h1, degp, src3, dst3, zrows)
  p2, hs2 = _sc_agg2(p1, hs1, dinv, b1, src3, dst3, zrows)
  return _tc_b(p2, hs2, dinv, W2, b2b, Wlin, blinb)


# speedup vs baseline: 63.5221x; 1.0035x over previous
import jax
import jax.numpy as jnp
from jax import lax
from jax.experimental import pallas as pl
from jax.experimental.pallas import tpu as pltpu
from jax.experimental.pallas import tpu_sc as plsc

N = 10000
E = 320000
F_IN = 128
F_MID = 16

NC = 2
NS = 16
NW = NC * NS
CHUNK = 512
CPW = 20
NBUF = 4
E_PAD = NW * CPW * CHUNK
N_PAD = 10240
ROWS_PT = N_PAD // NS

_mesh = plsc.VectorSubcoreMesh(core_axis_name="c", subcore_axis_name="s")
_sc_params = pltpu.CompilerParams(use_tc_tiling_on_sc=False,
                                 needs_layout_passes=False)


def _rsqrt16(d):
  yi = jnp.int32(0x5F3759DF) - (plsc.bitcast(d, jnp.int32) >> 1)
  y = plsc.bitcast(yi, jnp.float32)
  for _ in range(3):
    y = y * (1.5 - 0.5 * d * y * y)
  return y


def _edge_loop(tbl_s, acc, src_v, dst_v, bufs, semg, sems):
  assert CPW % 2 == 0
  rounds = CPW // 2
  rings = [(0, 1), (2, 3)]

  def gather(j, b):
    pltpu.async_copy(tbl_s.at[src_v.at[j]], bufs[b], semg[b])

  def gwait(b):
    pltpu.make_async_copy(tbl_s.at[src_v.at[0]], bufs[b], semg[b]).wait()

  def scat(j, b):
    pltpu.async_copy(bufs[b], acc.at[dst_v.at[j]], sems[b], add=True)

  def swait(b):
    pltpu.make_async_copy(bufs[b], acc.at[dst_v.at[0]], sems[b]).wait()

  for b in range(NBUF):
    gather(b, b)
  for r in range(rounds):
    ring = rings[r % 2]
    for k in (0, 1):
      gwait(ring[k])
    for k in (0, 1):
      scat(2 * r + k, ring[k])
    if r >= 1:
      prev = rings[(r - 1) % 2]
      for k in (0, 1):
        swait(prev[k])
        if 2 * (r + 1) + k < CPW:
          gather(2 * (r + 1) + k, prev[k])
  for k in (0, 1):
    swait(rings[(rounds - 1) % 2][k])


def _sc_deg_kernel():
  def body(ones_h, dst_h, zrows_h, out_h, dst_v, ones_v, sem, acc):
    c = lax.axis_index("c")
    s = lax.axis_index("s")
    wid = s * NC + c
    a0 = pltpu.async_copy(zrows_h, acc.at[pl.ds(s * ROWS_PT, ROWS_PT)], sem)
    a1 = pltpu.async_copy(dst_h.at[wid], dst_v, sem)
    a2 = pltpu.async_copy(ones_h, ones_v, sem)
    a0.wait(); a1.wait(); a2.wait()
    plsc.subcore_barrier()

    @pl.loop(0, CPW)
    def _(j):
      pltpu.async_copy(ones_v, acc.at[dst_v.at[j]], sem, add=True)

    @pl.loop(0, CPW)
    def _(j):
      pltpu.make_async_copy(ones_v, acc.at[dst_v.at[0]], sem).wait()

    plsc.subcore_barrier()
    pltpu.sync_copy(acc.at[pl.ds(s * ROWS_PT, ROWS_PT)],
                    out_h.at[c, pl.ds(s * ROWS_PT, ROWS_PT)])

  return pl.kernel(
      body,
      out_type=jax.ShapeDtypeStruct((NC, N_PAD, F_MID), jnp.float32),
      mesh=_mesh,
      scratch_types=[
          pltpu.VMEM((CPW, CHUNK), jnp.int32),
          pltpu.VMEM((CHUNK, F_MID), jnp.float32),
          pltpu.SemaphoreType.DMA,
          pltpu.VMEM_SHARED((N_PAD, F_MID), jnp.float32),
      ],
      compiler_params=_sc_params,
  )


def _agg_scratch(extra):
  return [
      pltpu.VMEM((CPW, CHUNK), jnp.int32),
      pltpu.VMEM((CPW, CHUNK), jnp.int32),
      pltpu.VMEM((ROWS_PT, F_MID), jnp.float32),
      pltpu.VMEM((ROWS_PT, F_MID), jnp.float32),
      pltpu.VMEM((ROWS_PT, F_MID), jnp.float32),
      pltpu.VMEM((ROWS_PT, F_MID), jnp.float32),
      pltpu.VMEM_SHARED((N_PAD, F_MID), jnp.float32),
      pltpu.VMEM_SHARED((N_PAD, F_MID), jnp.float32),
  ] + extra + [pltpu.VMEM((CHUNK, F_MID), jnp.float32) for _ in range(NBUF)] \
    + [pltpu.SemaphoreType.DMA for _ in range(2 * NBUF)]


def _sc_agg1_kernel():
  def body(h1_h, degp_h, src_h, dst_h, zrows_h,
           p_h, hs_out_h, dv_out_h,
           src_v, dst_v, t0, t1, th, tdv, acc, tbl_s, *rest):
    bufs, semg, sems = rest[:NBUF], rest[NBUF:2 * NBUF], rest[2 * NBUF:]
    c = lax.axis_index("c")
    s = lax.axis_index("s")
    wid = s * NC + c
    rsl = pl.ds(s * ROWS_PT, ROWS_PT)
    a_acc = pltpu.async_copy(zrows_h, acc.at[rsl], semg[0])
    a_src = pltpu.async_copy(src_h.at[wid], src_v, semg[1])
    a_dst = pltpu.async_copy(dst_h.at[wid], dst_v, semg[2])
    a_t0 = pltpu.async_copy(degp_h.at[0, rsl], t0, semg[3])
    a_t1 = pltpu.async_copy(degp_h.at[1, rsl], t1, sems[0])
    a_th = pltpu.async_copy(h1_h.at[rsl], th, sems[1])
    a_t0.wait(); a_t1.wait(); a_th.wait()

    @plsc.parallel_loop(0, ROWS_PT, unroll=8)
    def _(i):
      dinv = _rsqrt16(t0[i, :] + t1[i, :] + 1.0)
      tdv[i, :] = dinv
      th[i, :] = dinv * th[i, :]

    a_tbl = pltpu.async_copy(th, tbl_s.at[rsl], sems[2])

    @pl.when(c == 0)
    def _():
      pltpu.async_copy(th, hs_out_h.at[rsl], sems[3])
      pltpu.async_copy(tdv, dv_out_h.at[rsl], semg[0])

    a_acc.wait(); a_src.wait(); a_dst.wait(); a_tbl.wait()

    @pl.when(c == 0)
    def _():
      pltpu.make_async_copy(th, hs_out_h.at[rsl], sems[3]).wait()
      pltpu.make_async_copy(tdv, dv_out_h.at[rsl], semg[0]).wait()

    plsc.subcore_barrier()
    _edge_loop(tbl_s, acc, src_v, dst_v, bufs, semg, sems)
    plsc.subcore_barrier()
    pltpu.sync_copy(acc.at[rsl], p_h.at[c, rsl])

  return pl.kernel(
      body,
      out_type=(jax.ShapeDtypeStruct((NC, N_PAD, F_MID), jnp.float32),
                jax.ShapeDtypeStruct((N_PAD, F_MID), jnp.float32),
                jax.ShapeDtypeStruct((N_PAD, F_MID), jnp.float32)),
      mesh=_mesh,
      scratch_types=_agg_scratch([]),
      compiler_params=_sc_params,
  )


def _sc_agg2_kernel():
  def body(p1_h, hs1_h, dv_h, b1_h, src_h, dst_h, zrows_h,
           p_h, hs_out_h,
           src_v, dst_v, t0, t1, th, tdv, acc, tbl_s, tb, *rest):
    bufs, semg, sems = rest[:NBUF], rest[NBUF:2 * NBUF], rest[2 * NBUF:]
    c = lax.axis_index("c")
    s = lax.axis_index("s")
    wid = s * NC + c
    rsl = pl.ds(s * ROWS_PT, ROWS_PT)
    a_acc = pltpu.async_copy(zrows_h, acc.at[rsl], semg[0])
    a_src = pltpu.async_copy(src_h.at[wid], src_v, semg[1])
    a_dst = pltpu.async_copy(dst_h.at[wid], dst_v, semg[2])
    a_t0 = pltpu.async_copy(p1_h.at[0, rsl], t0, semg[3])
    a_t1 = pltpu.async_copy(p1_h.at[1, rsl], t1, sems[0])
    a_th = pltpu.async_copy(hs1_h.at[rsl], th, sems[1])
    a_dv = pltpu.async_copy(dv_h.at[rsl], tdv, sems[2])
    a_tb = pltpu.async_copy(b1_h, tb, sems[3])
    a_t0.wait(); a_t1.wait(); a_th.wait(); a_dv.wait(); a_tb.wait()

    @plsc.parallel_loop(0, ROWS_PT, unroll=8)
    def _(i):
      dinv = tdv[i, :]
      z = dinv * (t0[i, :] + t1[i, :] + th[i, :]) + tb[:]
      th[i, :] = dinv * jnp.maximum(z, 0.0)

    a_tbl = pltpu.async_copy(th, tbl_s.at[rsl], sems[2])

    @pl.when(c == 0)
    def _():
      pltpu.async_copy(th, hs_out_h.at[rsl], sems[3])

    a_acc.wait(); a_src.wait(); a_dst.wait(); a_tbl.wait()

    @pl.when(c == 0)
    def _():
      pltpu.make_async_copy(th, hs_out_h.at[rsl], sems[3]).wait()

    plsc.subcore_barrier()
    _edge_loop(tbl_s, acc, src_v, dst_v, bufs, semg, sems)
    plsc.subcore_barrier()
    pltpu.sync_copy(acc.at[rsl], p_h.at[c, rsl])

  return pl.kernel(
      body,
      out_type=(jax.ShapeDtypeStruct((NC, N_PAD, F_MID), jnp.float32),
                jax.ShapeDtypeStruct((N_PAD, F_MID), jnp.float32)),
      mesh=_mesh,
      scratch_types=_agg_scratch([pltpu.VMEM((F_MID,), jnp.float32)]),
      compiler_params=_sc_params,
  )


_sc_deg = _sc_deg_kernel()
_sc_agg1 = _sc_agg1_kernel()
_sc_agg2 = _sc_agg2_kernel()

_RB = 2000
_GRID = N // _RB


def _tc_a(x, w1):
  def body(x_ref, w_ref, o_ref):
    o_ref[...] = jnp.dot(x_ref[...], w_ref[...],
                         preferred_element_type=jnp.float32)

  return pl.pallas_call(
      body,
      grid=(_GRID,),
      in_specs=[
          pl.BlockSpec((_RB, F_IN), lambda i: (i, 0)),
          pl.BlockSpec((F_IN, F_MID), lambda i: (0, 0)),
      ],
      out_specs=pl.BlockSpec((_RB, F_MID), lambda i: (i, 0)),
      out_shape=jax.ShapeDtypeStruct((N_PAD, F_MID), jnp.float32),
  )(x, w1)


def _tc_b(p, hs2, dinv, w2, b2b, wlin, blinb):
  def body(p_ref, hs_ref, dv_ref, w2_ref, b2_ref, wl_ref, bl_ref, o_ref):
    agg = dv_ref[...] * (p_ref[0] + p_ref[1] + hs_ref[...])
    h2 = jnp.dot(agg, w2_ref[...], preferred_element_type=jnp.float32)
    h2 = jnp.maximum(h2 + b2_ref[0:1, :], 0.0)
    o_ref[...] = (jnp.dot(h2, wl_ref[...], preferred_element_type=jnp.float32)
                  + bl_ref[0:1, 0:1])

  return pl.pallas_call(
      body,
      grid=(_GRID,),
      in_specs=[
          pl.BlockSpec((NC, _RB, F_MID), lambda i: (0, i, 0)),
          pl.BlockSpec((_RB, F_MID), lambda i: (i, 0)),
          pl.BlockSpec((_RB, F_MID), lambda i: (i, 0)),
          pl.BlockSpec((F_MID, F_IN), lambda i: (0, 0)),
          pl.BlockSpec((8, F_IN), lambda i: (0, 0)),
          pl.BlockSpec((F_IN, 1), lambda i: (0, 0)),
          pl.BlockSpec((8, 8), lambda i: (0, 0)),
      ],
      out_specs=pl.BlockSpec((_RB, 1), lambda i: (i, 0)),
      out_shape=jax.ShapeDtypeStruct((N, 1), jnp.float32),
  )(p, hs2, dinv, w2, b2b, wlin, blinb)


@jax.jit
def kernel(x, edge_index, W1, b1, W2, b2, Wlin, blin):
  src = edge_index[0].astype(jnp.int32)
  dst = edge_index[1].astype(jnp.int32)
  pad = E_PAD - E
  src3 = jnp.concatenate(
      [src, jnp.zeros((pad,), jnp.int32)]).reshape(NW, CPW, CHUNK)
  dst3 = jnp.concatenate(
      [dst, jnp.full((pad,), N_PAD - 1, jnp.int32)]).reshape(NW, CPW, CHUNK)
  ones_rows = jnp.ones((CHUNK, F_MID), jnp.float32)
  zrows = jnp.zeros((ROWS_PT, F_MID), jnp.float32)
  b2b = jnp.broadcast_to(b2.reshape(1, F_IN), (8, F_IN))
  blinb = jnp.broadcast_to(blin.reshape(1, 1), (8, 8))

  h1 = _tc_a(x, W1)
  degp = _sc_deg(ones_rows, dst3, zrows)
  p1, hs1, dinv = _sc_agg1(h1, degp, src3, dst3, zrows)
  p2, hs2 = _sc_agg2(p1, hs1, dinv, b1, src3, dst3, zrows)
  return _tc_b(p2, hs2, dinv, W2, b2b, Wlin, blinb)


# speedup vs baseline: 63.5567x; 1.0005x over previous
import jax
import jax.numpy as jnp
from jax import lax
from jax.experimental import pallas as pl
from jax.experimental.pallas import tpu as pltpu
from jax.experimental.pallas import tpu_sc as plsc

N = 10000
E = 320000
F_IN = 128
F_MID = 16

NC = 2
NS = 16
NW = NC * NS
CHUNK = 512
CPW = 20
NBUF = 4
E_PAD = NW * CPW * CHUNK
N_PAD = 10240
ROWS_PT = N_PAD // NS

_mesh = plsc.VectorSubcoreMesh(core_axis_name="c", subcore_axis_name="s")
_sc_params = pltpu.CompilerParams(use_tc_tiling_on_sc=False,
                                 needs_layout_passes=False)


def _rsqrt16(d):
  yi = jnp.int32(0x5F3759DF) - (plsc.bitcast(d, jnp.int32) >> 1)
  y = plsc.bitcast(yi, jnp.float32)
  for _ in range(3):
    y = y * (1.5 - 0.5 * d * y * y)
  return y


def _edge_loop(tbl_s, acc, src_v, dst_v, bufs, semg, sems):
  assert CPW % 2 == 0
  rounds = CPW // 2
  rings = [(0, 1), (2, 3)]

  def gather(j, b):
    pltpu.async_copy(tbl_s.at[src_v.at[j]], bufs[b], semg[b])

  def gwait(b):
    pltpu.make_async_copy(tbl_s.at[src_v.at[0]], bufs[b], semg[b]).wait()

  def scat(j, b):
    pltpu.async_copy(bufs[b], acc.at[dst_v.at[j]], sems[b], add=True)

  def swait(b):
    pltpu.make_async_copy(bufs[b], acc.at[dst_v.at[0]], sems[b]).wait()

  for b in range(NBUF):
    gather(b, b)
  for r in range(rounds):
    ring = rings[r % 2]
    for k in (0, 1):
      gwait(ring[k])
    for k in (0, 1):
      scat(2 * r + k, ring[k])
    if r >= 1:
      prev = rings[(r - 1) % 2]
      for k in (0, 1):
        swait(prev[k])
        if 2 * (r + 1) + k < CPW:
          gather(2 * (r + 1) + k, prev[k])
  for k in (0, 1):
    swait(rings[(rounds - 1) % 2][k])


def _sc_deg_kernel():
  def body(ones_h, dst_h, zrows_h, out_h, dst_v, ones_v, sem, acc):
    c = lax.axis_index("c")
    s = lax.axis_index("s")
    wid = s * NC + c
    a0 = pltpu.async_copy(zrows_h, acc.at[pl.ds(s * ROWS_PT, ROWS_PT)], sem)
    a1 = pltpu.async_copy(dst_h.at[wid], dst_v, sem)
    a2 = pltpu.async_copy(ones_h, ones_v, sem)
    a0.wait(); a1.wait(); a2.wait()
    plsc.subcore_barrier()

    @pl.loop(0, CPW)
    def _(j):
      pltpu.async_copy(ones_v, acc.at[dst_v.at[j]], sem, add=True)

    @pl.loop(0, CPW)
    def _(j):
      pltpu.make_async_copy(ones_v, acc.at[dst_v.at[0]], sem).wait()

    plsc.subcore_barrier()
    pltpu.sync_copy(acc.at[pl.ds(s * ROWS_PT, ROWS_PT)],
                    out_h.at[c, pl.ds(s * ROWS_PT, ROWS_PT)])

  return pl.kernel(
      body,
      out_type=jax.ShapeDtypeStruct((NC, N_PAD, F_MID), jnp.float32),
      mesh=_mesh,
      scratch_types=[
          pltpu.VMEM((CPW, CHUNK), jnp.int32),
          pltpu.VMEM((CHUNK, F_MID), jnp.float32),
          pltpu.SemaphoreType.DMA,
          pltpu.VMEM_SHARED((N_PAD, F_MID), jnp.float32),
      ],
      compiler_params=_sc_params,
  )


def _agg_scratch(extra):
  return [
      pltpu.VMEM((CPW, CHUNK), jnp.int32),
      pltpu.VMEM((CPW, CHUNK), jnp.int32),
      pltpu.VMEM((ROWS_PT, F_MID), jnp.float32),
      pltpu.VMEM((ROWS_PT, F_MID), jnp.float32),
      pltpu.VMEM((ROWS_PT, F_MID), jnp.float32),
      pltpu.VMEM((ROWS_PT, F_MID), jnp.float32),
      pltpu.VMEM_SHARED((N_PAD, F_MID), jnp.float32),
      pltpu.VMEM_SHARED((N_PAD, F_MID), jnp.float32),
  ] + extra + [pltpu.VMEM((CHUNK, F_MID), jnp.float32) for _ in range(NBUF)] \
    + [pltpu.SemaphoreType.DMA for _ in range(2 * NBUF)]


def _sc_agg1_kernel():
  def body(h1_h, degp_h, src_h, dst_h, zrows_h,
           p_h, hs_out_h, dv_out_h,
           src_v, dst_v, t0, t1, th, tdv, acc, tbl_s, *rest):
    bufs, semg, sems = rest[:NBUF], rest[NBUF:2 * NBUF], rest[2 * NBUF:]
    c = lax.axis_index("c")
    s = lax.axis_index("s")
    wid = s * NC + c
    rsl = pl.ds(s * ROWS_PT, ROWS_PT)
    a_acc = pltpu.async_copy(zrows_h, acc.at[rsl], semg[0])
    a_src = pltpu.async_copy(src_h.at[wid], src_v, semg[1])
    a_dst = pltpu.async_copy(dst_h.at[wid], dst_v, semg[2])
    a_t0 = pltpu.async_copy(degp_h.at[0, rsl], t0, semg[3])
    a_t1 = pltpu.async_copy(degp_h.at[1, rsl], t1, sems[0])
    a_th = pltpu.async_copy(h1_h.at[rsl], th, sems[1])
    a_t0.wait(); a_t1.wait(); a_th.wait()

    @plsc.parallel_loop(0, ROWS_PT, unroll=8)
    def _(i):
      dinv = _rsqrt16(t0[i, :] + t1[i, :] + 1.0)
      tdv[i, :] = dinv
      th[i, :] = dinv * th[i, :]

    a_tbl = pltpu.async_copy(th, tbl_s.at[rsl], sems[2])

    @pl.when(c == 0)
    def _():
      pltpu.async_copy(th, hs_out_h.at[rsl], sems[3])
      pltpu.async_copy(tdv, dv_out_h.at[rsl], semg[0])

    a_acc.wait(); a_src.wait(); a_dst.wait(); a_tbl.wait()

    @pl.when(c == 0)
    def _():
      pltpu.make_async_copy(th, hs_out_h.at[rsl], sems[3]).wait()
      pltpu.make_async_copy(tdv, dv_out_h.at[rsl], semg[0]).wait()

    plsc.subcore_barrier()
    _edge_loop(tbl_s, acc, src_v, dst_v, bufs, semg, sems)
    plsc.subcore_barrier()
    pltpu.sync_copy(acc.at[rsl], p_h.at[c, rsl])

  return pl.kernel(
      body,
      out_type=(jax.ShapeDtypeStruct((NC, N_PAD, F_MID), jnp.float32),
                jax.ShapeDtypeStruct((N_PAD, F_MID), jnp.float32),
                jax.ShapeDtypeStruct((N_PAD, F_MID), jnp.float32)),
      mesh=_mesh,
      scratch_types=_agg_scratch([]),
      compiler_params=_sc_params,
  )


def _sc_agg2_kernel():
  def body(p1_h, hs1_h, dv_h, b1_h, src_h, dst_h, zrows_h,
           p_h, hs_out_h,
           src_v, dst_v, t0, t1, th, tdv, acc, tbl_s, tb, *rest):
    bufs, semg, sems = rest[:NBUF], rest[NBUF:2 * NBUF], rest[2 * NBUF:]
    c = lax.axis_index("c")
    s = lax.axis_index("s")
    wid = s * NC + c
    rsl = pl.ds(s * ROWS_PT, ROWS_PT)
    a_acc = pltpu.async_copy(zrows_h, acc.at[rsl], semg[0])
    a_src = pltpu.async_copy(src_h.at[wid], src_v, semg[1])
    a_dst = pltpu.async_copy(dst_h.at[wid], dst_v, semg[2])
    a_t0 = pltpu.async_copy(p1_h.at[0, rsl], t0, semg[3])
    a_t1 = pltpu.async_copy(p1_h.at[1, rsl], t1, sems[0])
    a_th = pltpu.async_copy(hs1_h.at[rsl], th, sems[1])
    a_dv = pltpu.async_copy(dv_h.at[rsl], tdv, sems[2])
    a_tb = pltpu.async_copy(b1_h, tb, sems[3])
    a_t0.wait(); a_t1.wait(); a_th.wait(); a_dv.wait(); a_tb.wait()

    @plsc.parallel_loop(0, ROWS_PT, unroll=8)
    def _(i):
      dinv = tdv[i, :]
      z = dinv * (t0[i, :] + t1[i, :] + th[i, :]) + tb[:]
      th[i, :] = dinv * jnp.maximum(z, 0.0)

    a_tbl = pltpu.async_copy(th, tbl_s.at[rsl], sems[2])

    @pl.when(c == 0)
    def _():
      pltpu.async_copy(th, hs_out_h.at[rsl], sems[3])

    a_acc.wait(); a_src.wait(); a_dst.wait(); a_tbl.wait()

    @pl.when(c == 0)
    def _():
      pltpu.make_async_copy(th, hs_out_h.at[rsl], sems[3]).wait()

    plsc.subcore_barrier()
    _edge_loop(tbl_s, acc, src_v, dst_v, bufs, semg, sems)
    plsc.subcore_barrier()
    pltpu.sync_copy(acc.at[rsl], p_h.at[c, rsl])

  return pl.kernel(
      body,
      out_type=(jax.ShapeDtypeStruct((NC, N_PAD, F_MID), jnp.float32),
                jax.ShapeDtypeStruct((N_PAD, F_MID), jnp.float32)),
      mesh=_mesh,
      scratch_types=_agg_scratch([pltpu.VMEM((F_MID,), jnp.float32)]),
      compiler_params=_sc_params,
  )


_sc_deg = _sc_deg_kernel()
_sc_agg1 = _sc_agg1_kernel()
_sc_agg2 = _sc_agg2_kernel()

_RB = 10000
_GRID = N // _RB


def _tc_a(x, w1):
  def body(x_ref, w_ref, o_ref):
    o_ref[...] = jnp.dot(x_ref[...], w_ref[...],
                         preferred_element_type=jnp.float32)

  return pl.pallas_call(
      body,
      grid=(_GRID,),
      in_specs=[
          pl.BlockSpec((_RB, F_IN), lambda i: (i, 0)),
          pl.BlockSpec((F_IN, F_MID), lambda i: (0, 0)),
      ],
      out_specs=pl.BlockSpec((_RB, F_MID), lambda i: (i, 0)),
      out_shape=jax.ShapeDtypeStruct((N_PAD, F_MID), jnp.float32),
  )(x, w1)


def _tc_b(p, hs2, dinv, w2, b2b, wlin, blinb):
  def body(p_ref, hs_ref, dv_ref, w2_ref, b2_ref, wl_ref, bl_ref, o_ref):
    agg = dv_ref[...] * (p_ref[0] + p_ref[1] + hs_ref[...])
    h2 = jnp.dot(agg, w2_ref[...], preferred_element_type=jnp.float32)
    h2 = jnp.maximum(h2 + b2_ref[0:1, :], 0.0)
    o_ref[...] = (jnp.dot(h2, wl_ref[...], preferred_element_type=jnp.float32)
                  + bl_ref[0:1, 0:1])

  return pl.pallas_call(
      body,
      grid=(_GRID,),
      in_specs=[
          pl.BlockSpec((NC, _RB, F_MID), lambda i: (0, i, 0)),
          pl.BlockSpec((_RB, F_MID), lambda i: (i, 0)),
          pl.BlockSpec((_RB, F_MID), lambda i: (i, 0)),
          pl.BlockSpec((F_MID, F_IN), lambda i: (0, 0)),
          pl.BlockSpec((8, F_IN), lambda i: (0, 0)),
          pl.BlockSpec((F_IN, 1), lambda i: (0, 0)),
          pl.BlockSpec((8, 8), lambda i: (0, 0)),
      ],
      out_specs=pl.BlockSpec((_RB, 1), lambda i: (i, 0)),
      out_shape=jax.ShapeDtypeStruct((N, 1), jnp.float32),
  )(p, hs2, dinv, w2, b2b, wlin, blinb)


@jax.jit
def kernel(x, edge_index, W1, b1, W2, b2, Wlin, blin):
  src = edge_index[0].astype(jnp.int32)
  dst = edge_index[1].astype(jnp.int32)
  pad = E_PAD - E
  src3 = jnp.concatenate(
      [src, jnp.zeros((pad,), jnp.int32)]).reshape(NW, CPW, CHUNK)
  dst3 = jnp.concatenate(
      [dst, jnp.full((pad,), N_PAD - 1, jnp.int32)]).reshape(NW, CPW, CHUNK)
  ones_rows = jnp.ones((CHUNK, F_MID), jnp.float32)
  zrows = jnp.zeros((ROWS_PT, F_MID), jnp.float32)
  b2b = jnp.broadcast_to(b2.reshape(1, F_IN), (8, F_IN))
  blinb = jnp.broadcast_to(blin.reshape(1, 1), (8, 8))

  h1 = _tc_a(x, W1)
  degp = _sc_deg(ones_rows, dst3, zrows)
  p1, hs1, dinv = _sc_agg1(h1, degp, src3, dst3, zrows)
  p2, hs2 = _sc_agg2(p1, hs1, dinv, b1, src3, dst3, zrows)
  return _tc_b(p2, hs2, dinv, W2, b2b, Wlin, blinb)
